# baseline (speedup 1.0000x reference)
import functools

import jax
import jax.numpy as jnp
from jax import lax
from jax.experimental import pallas as pl
from jax.experimental.pallas import tpu as pltpu
from jax.experimental.pallas import tpu_sc as plsc

_D_IN = 2048
_PKM = 1000
_PAD = 1024
_K = 32
_N_TOK = 8192
_BLK = 256
_NEG_PAD = -1e30
_DEAD = -3e38

_NC = 2
_NS = 16
_L = 16
_NW = _NC * _NS
_RPW = _N_TOK // _NW
_RB = 16
_NBATCH = _RPW // _RB

_AB = [(a, b) for a in range(_K) for b in range(_K) if (a + 1) * (b + 1) <= _K]
_NCAND = 128
_NSPLIT = 2


def _mm_body(x_ref, w_ref, b_ref, h_ref, t_ref):
    h = jnp.dot(x_ref[...], w_ref[...], preferred_element_type=jnp.float32)
    h = h + b_ref[...]
    h_ref[...] = h

    def thresh(v):
        w = _PAD
        while w > _K:
            w //= 2
            v = jnp.maximum(v[:, :w], v[:, w:])
        return jnp.min(v, axis=1, keepdims=True)

    t_ref[...] = jnp.concatenate(
        [thresh(h[:, :_PAD]), thresh(h[:, _PAD:])], axis=1)


def _matmul(x, wpt, bp):
    ntok = x.shape[0]
    fixed = lambda i: (0, 0)
    return pl.pallas_call(
        _mm_body,
        grid=(ntok // _BLK,),
        in_specs=[
            pl.BlockSpec((_BLK, _D_IN), lambda i: (i, 0)),
            pl.BlockSpec((_D_IN, 2 * _PAD), fixed),
            pl.BlockSpec((1, 2 * _PAD), fixed),
        ],
        out_specs=[
            pl.BlockSpec((_BLK, 2 * _PAD), lambda i: (i, 0)),
            pl.BlockSpec((_BLK, 2), lambda i: (i, 0)),
        ],
        out_shape=[
            jax.ShapeDtypeStruct((ntok, 2 * _PAD), jnp.float32),
            jax.ShapeDtypeStruct((ntok, 2), jnp.float32),
        ],
        compiler_params=pltpu.CompilerParams(
            dimension_semantics=("parallel",),
        ),
    )(x, wpt, bp)


def _sortkv(keys, vals):
    return plsc.sort_key_val(keys, vals, descending=True)


def _minmax_kv(ak, av, bk, bv):
    m = ak >= bk
    hk = jnp.where(m, ak, bk)
    hv = jnp.where(m, av, bv)
    lk = jnp.where(m, bk, ak)
    lv = jnp.where(m, bv, av)
    return hk, hv, lk, lv


def _merge16_full(c1k, c1v, c2k, c2v):
    r2k = lax.rev(c2k, (0,))
    r2v = lax.rev(c2v, (0,))
    hk, hv, lk, lv = _minmax_kv(c1k, c1v, r2k, r2v)
    t1k, t1v = _sortkv(hk, hv)
    t2k, t2v = _sortkv(lk, lv)
    return t1k, t1v, t2k, t2v


def _merge32_top(r1k, r1v, r2k, r2v, t1k, t1v, t2k, t2v):
    x1k, x1v, _, _ = _minmax_kv(r1k, r1v, lax.rev(t2k, (0,)),
                                lax.rev(t2v, (0,)))
    x2k, x2v, _, _ = _minmax_kv(r2k, r2v, lax.rev(t1k, (0,)),
                                lax.rev(t1v, (0,)))
    y1k, y1v, y2k, y2v = _minmax_kv(x1k, x1v, x2k, x2v)
    r1k, r1v = _sortkv(y1k, y1v)
    r2k, r2v = _sortkv(y2k, y2v)
    return r1k, r2k, r1v, r2v


def _make_sc_body(rpw, nbatch):
  def _sc_body(h_hbm, t_hbm, at_hbm, bt_hbm, pv_hbm, ow_hbm, oi_hbm,
               hbuf, tbuf, sval, sidx, sval2, sidx2, w12, i12, atv, btv, pvv,
               wout, iout):
    wid = lax.axis_index("s") * _NC + lax.axis_index("c")
    row0 = wid * rpw
    pltpu.sync_copy(at_hbm, atv)
    pltpu.sync_copy(bt_hbm, btv)
    pltpu.sync_copy(pv_hbm, pvv)
    iota = lax.broadcasted_iota(jnp.int32, (_L,), 0)
    negv = jnp.full((_L,), _DEAD, jnp.float32)
    bigv = jnp.full((_L,), 1 << 30, jnp.int32)

    def compact_both(hb_base, t1, t2):
        def comp(j, c):
            oa, ob = c
            basea = hb_base + 2 * _L * j
            baseb = basea + _PAD
            va1 = hbuf[pl.ds(basea, _L)]
            va2 = hbuf[pl.ds(basea + _L, _L)]
            vb1 = hbuf[pl.ds(baseb, _L)]
            vb2 = hbuf[pl.ds(baseb + _L, _L)]
            ma1 = va1 >= t1
            ma2 = va2 >= t1
            mb1 = vb1 >= t2
            mb2 = vb2 >= t2
            ix = iota + 2 * _L * j
            plsc.store_compressed(sval.at[pl.ds(oa, _L)], va1, mask=ma1)
            plsc.store_compressed(sidx.at[pl.ds(oa, _L)], ix, mask=ma1)
            plsc.store_compressed(sval2.at[pl.ds(ob, _L)], vb1, mask=mb1)
            plsc.store_compressed(sidx2.at[pl.ds(ob, _L)], ix, mask=mb1)
            oa1 = oa + plsc.all_reduce_population_count(ma1)[0]
            ob1 = ob + plsc.all_reduce_population_count(mb1)[0]
            plsc.store_compressed(sval.at[pl.ds(oa1, _L)], va2, mask=ma2)
            plsc.store_compressed(sidx.at[pl.ds(oa1, _L)], ix + _L, mask=ma2)
            plsc.store_compressed(sval2.at[pl.ds(ob1, _L)], vb2, mask=mb2)
            plsc.store_compressed(sidx2.at[pl.ds(ob1, _L)], ix + _L,
                                  mask=mb2)
            return (oa1 + plsc.all_reduce_population_count(ma2)[0],
                    ob1 + plsc.all_reduce_population_count(mb2)[0])

        oa, ob = lax.fori_loop(0, _PAD // (2 * _L), comp,
                               (jnp.int32(0), jnp.int32(0)))
        sval[pl.ds(oa, _L)] = negv
        sidx[pl.ds(oa, _L)] = bigv
        sval[pl.ds(oa + _L, _L)] = negv
        sidx[pl.ds(oa + _L, _L)] = bigv
        sval2[pl.ds(ob, _L)] = negv
        sidx2[pl.ds(ob, _L)] = bigv
        sval2[pl.ds(ob + _L, _L)] = negv
        sidx2[pl.ds(ob + _L, _L)] = bigv
        return oa, ob

    def merge_surv(vref, iref, o):
        npair = (o + 2 * _L - 1) // (2 * _L)

        def mstep(j, c):
            r1k, r2k, r1v, r2v = c
            b = 2 * _L * j
            c1k, c1v = _sortkv(vref[pl.ds(b, _L)], iref[pl.ds(b, _L)])
            c2k, c2v = _sortkv(vref[pl.ds(b + _L, _L)],
                               iref[pl.ds(b + _L, _L)])
            t1k, t1v, t2k, t2v = _merge16_full(c1k, c1v, c2k, c2v)
            return _merge32_top(r1k, r1v, r2k, r2v, t1k, t1v, t2k, t2v)

        return lax.fori_loop(0, npair, mstep, (negv, negv, bigv, bigv))

    def row_body(r_glob):
        rl = r_glob % _RB
        hb_base = rl * (2 * _PAD)
        tv = tbuf[pl.ds(2 * rl, _L)]
        oa, ob = compact_both(hb_base, tv[0], tv[1])
        w1a, w1b, p1a, p1b = merge_surv(sval, sidx, oa)
        w2a, w2b, p2a, p2b = merge_surv(sval2, sidx2, ob)
        w12[pl.ds(0, _L)] = w1a
        w12[pl.ds(_L, _L)] = w1b
        w12[pl.ds(2 * _L, _L)] = w2a
        w12[pl.ds(3 * _L, _L)] = w2b
        i12[pl.ds(0, _L)] = p1a
        i12[pl.ds(_L, _L)] = p1b
        i12[pl.ds(2 * _L, _L)] = p2a
        i12[pl.ds(3 * _L, _L)] = p2b

        def cchunk(jj):
            ai = atv[pl.ds(_L * jj, _L)]
            bi = btv[pl.ds(_L * jj, _L)]
            pv = pvv[pl.ds(_L * jj, _L)]
            ga = plsc.load_gather(w12, [ai])
            gb = plsc.load_gather(w12, [bi])
            ia = plsc.load_gather(i12, [ai])
            ib = plsc.load_gather(i12, [bi])
            ck = jnp.maximum(ga + gb, 0.0) + pv
            cv = ia * _PKM + ib
            return _sortkv(ck, cv)

        r1, r2, v1, v2 = negv, negv, bigv, bigv
        for j in range(_NCAND // (2 * _L)):
            c1k, c1v = cchunk(2 * j)
            c2k, c2v = cchunk(2 * j + 1)
            t1k, t1v, t2k, t2v = _merge16_full(c1k, c1v, c2k, c2v)
            r1, r2, v1, v2 = _merge32_top(r1, v1, r2, v2,
                                          t1k, t1v, t2k, t2v)

        out_off = r_glob * _K
        wout[pl.ds(out_off, _L)] = r1
        wout[pl.ds(out_off + _L, _L)] = r2
        iout[pl.ds(out_off, _L)] = v1
        iout[pl.ds(out_off + _L, _L)] = v2

    def batch_body(b, _):
        pltpu.sync_copy(
            h_hbm.at[pl.ds((row0 + b * _RB) * (2 * _PAD), _RB * 2 * _PAD)],
            hbuf)
        pltpu.sync_copy(
            t_hbm.at[pl.ds((row0 + b * _RB) * 2, _RB * 2)],
            tbuf.at[pl.ds(0, _RB * 2)])

        def rloop(r, _2):
            row_body(b * _RB + r)
            return 0

        lax.fori_loop(0, _RB, rloop, 0)
        return 0

    lax.fori_loop(0, nbatch, batch_body, 0)
    pltpu.sync_copy(wout, ow_hbm.at[pl.ds(row0 * _K, rpw * _K)])
    pltpu.sync_copy(iout, oi_hbm.at[pl.ds(row0 * _K, rpw * _K)])
  return _sc_body


def _sc_topk(h_flat, t_flat, atab, btab, padv, ntok):
    rpw = ntok // _NW
    nbatch = rpw // _RB
    mesh = plsc.VectorSubcoreMesh(core_axis_name="c", subcore_axis_name="s",
                                  num_cores=_NC, num_subcores=_NS)
    f = pl.kernel(
        _make_sc_body(rpw, nbatch),
        out_type=(
            jax.ShapeDtypeStruct((ntok * _K,), jnp.float32),
            jax.ShapeDtypeStruct((ntok * _K,), jnp.int32),
        ),
        mesh=mesh,
        compiler_params=pltpu.CompilerParams(needs_layout_passes=False),
        scratch_types=[
            pltpu.VMEM((_RB * 2 * _PAD,), jnp.float32),
            pltpu.VMEM((_RB * 2 + _L,), jnp.float32),
            pltpu.VMEM((_PAD + 2 * _L,), jnp.float32),
            pltpu.VMEM((_PAD + 2 * _L,), jnp.int32),
            pltpu.VMEM((_PAD + 2 * _L,), jnp.float32),
            pltpu.VMEM((_PAD + 2 * _L,), jnp.int32),
            pltpu.VMEM((4 * _L,), jnp.float32),
            pltpu.VMEM((4 * _L,), jnp.int32),
            pltpu.VMEM((_NCAND,), jnp.int32),
            pltpu.VMEM((_NCAND,), jnp.int32),
            pltpu.VMEM((_NCAND,), jnp.float32),
            pltpu.VMEM((rpw * _K,), jnp.float32),
            pltpu.VMEM((rpw * _K,), jnp.int32),
        ],
    )
    return f(h_flat, t_flat, atab, btab, padv)


def _sc_tables():
    import numpy as np
    at = np.zeros((_NCAND,), np.int32)
    bt = np.zeros((_NCAND,), np.int32)
    pv = np.full((_NCAND,), _DEAD, np.float32)
    for j, (a, b) in enumerate(_AB):
        at[j] = a
        bt[j] = b + _K
        pv[j] = 0.0
    return jnp.asarray(at), jnp.asarray(bt), jnp.asarray(pv)


def kernel(x, W, b_lin, bias, k):
    del bias
    npad = _PAD - _PKM
    zrows = jnp.zeros((npad, _D_IN), W.dtype)
    wpt = jnp.concatenate([W[:_PKM], zrows, W[_PKM:], zrows], axis=0).T
    negs = jnp.full((npad,), _NEG_PAD, jnp.float32)
    bp = jnp.concatenate(
        [b_lin[:_PKM], negs, b_lin[_PKM:], negs]).reshape(1, 2 * _PAD)
    atab, btab, padv = _sc_tables()
    ws, is_ = [], []
    nchunk = _N_TOK // _NSPLIT
    for c in range(_NSPLIT):
        xc = x[c * nchunk:(c + 1) * nchunk]
        h, tt = _matmul(xc, wpt, bp)
        w_flat, i_flat = _sc_topk(h.reshape(-1), tt.reshape(-1),
                                  atab, btab, padv, nchunk)
        ws.append(w_flat.reshape(nchunk, _K))
        is_.append(i_flat.reshape(nchunk, _K))
    w = jnp.concatenate(ws, axis=0)
    i = jnp.concatenate(is_, axis=0)
    keep = jnp.asarray(k) == _K
    w = jnp.where(keep, w, jnp.zeros_like(w))
    i = jnp.where(keep, i, jnp.zeros_like(i))
    return w, i


# speedup vs baseline: 1.0942x; 1.0942x over previous
import functools

import jax
import jax.numpy as jnp
from jax import lax
from jax.experimental import pallas as pl
from jax.experimental.pallas import tpu as pltpu
from jax.experimental.pallas import tpu_sc as plsc

_D_IN = 2048
_PKM = 1000
_PAD = 1024
_K = 32
_N_TOK = 8192
_BLK = 256
_NEG_PAD = -1e30
_DEAD = -3e38

_NC = 2
_NS = 16
_L = 16
_NW = _NC * _NS
_RPW = _N_TOK // _NW
_RB = 16
_NBATCH = _RPW // _RB

_AB = [(a, b) for a in range(_K) for b in range(_K) if (a + 1) * (b + 1) <= _K]
_NCAND = 128
_NSPLIT = 1


def _mm_body(x_ref, w_ref, b_ref, h_ref, t_ref):
    h = jnp.dot(x_ref[...], w_ref[...], preferred_element_type=jnp.float32)
    h = h + b_ref[...]
    h_ref[...] = h

    def thresh(v):
        w = _PAD
        while w > _K:
            w //= 2
            v = jnp.maximum(v[:, :w], v[:, w:])
        return jnp.min(v, axis=1, keepdims=True)

    t_ref[...] = jnp.concatenate(
        [thresh(h[:, :_PAD]), thresh(h[:, _PAD:])], axis=1)


def _matmul(x, wpt, bp):
    ntok = x.shape[0]
    fixed = lambda i: (0, 0)
    return pl.pallas_call(
        _mm_body,
        grid=(ntok // _BLK,),
        in_specs=[
            pl.BlockSpec((_BLK, _D_IN), lambda i: (i, 0)),
            pl.BlockSpec((_D_IN, 2 * _PAD), fixed),
            pl.BlockSpec((1, 2 * _PAD), fixed),
        ],
        out_specs=[
            pl.BlockSpec((_BLK, 2 * _PAD), lambda i: (i, 0)),
            pl.BlockSpec((_BLK, 2), lambda i: (i, 0)),
        ],
        out_shape=[
            jax.ShapeDtypeStruct((ntok, 2 * _PAD), jnp.float32),
            jax.ShapeDtypeStruct((ntok, 2), jnp.float32),
        ],
        compiler_params=pltpu.CompilerParams(
            dimension_semantics=("parallel",),
        ),
    )(x, wpt, bp)


def _sortkv(keys, vals):
    return plsc.sort_key_val(keys, vals, descending=True)


def _minmax_kv(ak, av, bk, bv):
    m = ak >= bk
    hk = jnp.where(m, ak, bk)
    hv = jnp.where(m, av, bv)
    lk = jnp.where(m, bk, ak)
    lv = jnp.where(m, bv, av)
    return hk, hv, lk, lv


def _merge16_full(c1k, c1v, c2k, c2v):
    r2k = lax.rev(c2k, (0,))
    r2v = lax.rev(c2v, (0,))
    hk, hv, lk, lv = _minmax_kv(c1k, c1v, r2k, r2v)
    t1k, t1v = _sortkv(hk, hv)
    t2k, t2v = _sortkv(lk, lv)
    return t1k, t1v, t2k, t2v


def _merge32_top(r1k, r1v, r2k, r2v, t1k, t1v, t2k, t2v):
    x1k, x1v, _, _ = _minmax_kv(r1k, r1v, lax.rev(t2k, (0,)),
                                lax.rev(t2v, (0,)))
    x2k, x2v, _, _ = _minmax_kv(r2k, r2v, lax.rev(t1k, (0,)),
                                lax.rev(t1v, (0,)))
    y1k, y1v, y2k, y2v = _minmax_kv(x1k, x1v, x2k, x2v)
    r1k, r1v = _sortkv(y1k, y1v)
    r2k, r2v = _sortkv(y2k, y2v)
    return r1k, r2k, r1v, r2v


def _make_sc_body(rpw, nbatch):
  def _sc_body(h_hbm, t_hbm, at_hbm, bt_hbm, pv_hbm, ow_hbm, oi_hbm,
               hbuf, tbuf, sval, sidx, sval2, sidx2, w12, i12, atv, btv, pvv,
               wout, iout):
    wid = lax.axis_index("s") * _NC + lax.axis_index("c")
    row0 = wid * rpw
    pltpu.sync_copy(at_hbm, atv)
    pltpu.sync_copy(bt_hbm, btv)
    pltpu.sync_copy(pv_hbm, pvv)
    iota = lax.broadcasted_iota(jnp.int32, (_L,), 0)
    negv = jnp.full((_L,), _DEAD, jnp.float32)
    bigv = jnp.full((_L,), 1 << 30, jnp.int32)

    def compact_both(hb_base, t1, t2):
        def comp(j, c):
            oa, ob = c
            basea = hb_base + 2 * _L * j
            baseb = basea + _PAD
            va1 = hbuf[pl.ds(basea, _L)]
            va2 = hbuf[pl.ds(basea + _L, _L)]
            vb1 = hbuf[pl.ds(baseb, _L)]
            vb2 = hbuf[pl.ds(baseb + _L, _L)]
            ma1 = va1 >= t1
            ma2 = va2 >= t1
            mb1 = vb1 >= t2
            mb2 = vb2 >= t2
            ix = iota + 2 * _L * j
            plsc.store_compressed(sval.at[pl.ds(oa, _L)], va1, mask=ma1)
            plsc.store_compressed(sidx.at[pl.ds(oa, _L)], ix, mask=ma1)
            plsc.store_compressed(sval2.at[pl.ds(ob, _L)], vb1, mask=mb1)
            plsc.store_compressed(sidx2.at[pl.ds(ob, _L)], ix, mask=mb1)
            oa1 = oa + plsc.all_reduce_population_count(ma1)[0]
            ob1 = ob + plsc.all_reduce_population_count(mb1)[0]
            plsc.store_compressed(sval.at[pl.ds(oa1, _L)], va2, mask=ma2)
            plsc.store_compressed(sidx.at[pl.ds(oa1, _L)], ix + _L, mask=ma2)
            plsc.store_compressed(sval2.at[pl.ds(ob1, _L)], vb2, mask=mb2)
            plsc.store_compressed(sidx2.at[pl.ds(ob1, _L)], ix + _L,
                                  mask=mb2)
            return (oa1 + plsc.all_reduce_population_count(ma2)[0],
                    ob1 + plsc.all_reduce_population_count(mb2)[0])

        oa, ob = lax.fori_loop(0, _PAD // (2 * _L), comp,
                               (jnp.int32(0), jnp.int32(0)))
        sval[pl.ds(oa, _L)] = negv
        sidx[pl.ds(oa, _L)] = bigv
        sval[pl.ds(oa + _L, _L)] = negv
        sidx[pl.ds(oa + _L, _L)] = bigv
        sval2[pl.ds(ob, _L)] = negv
        sidx2[pl.ds(ob, _L)] = bigv
        sval2[pl.ds(ob + _L, _L)] = negv
        sidx2[pl.ds(ob + _L, _L)] = bigv
        return oa, ob

    def merge_surv(vref, iref, o):
        npair = (o + 2 * _L - 1) // (2 * _L)

        def mstep(j, c):
            r1k, r2k, r1v, r2v = c
            b = 2 * _L * j
            c1k, c1v = _sortkv(vref[pl.ds(b, _L)], iref[pl.ds(b, _L)])
            c2k, c2v = _sortkv(vref[pl.ds(b + _L, _L)],
                               iref[pl.ds(b + _L, _L)])
            t1k, t1v, t2k, t2v = _merge16_full(c1k, c1v, c2k, c2v)
            return _merge32_top(r1k, r1v, r2k, r2v, t1k, t1v, t2k, t2v)

        return lax.fori_loop(0, npair, mstep, (negv, negv, bigv, bigv))

    def row_body(r_glob):
        rl = r_glob % _RB
        hb_base = rl * (2 * _PAD)
        tv = tbuf[pl.ds(2 * rl, _L)]
        oa, ob = compact_both(hb_base, tv[0], tv[1])
        w1a, w1b, p1a, p1b = merge_surv(sval, sidx, oa)
        w2a, w2b, p2a, p2b = merge_surv(sval2, sidx2, ob)
        w12[pl.ds(0, _L)] = w1a
        w12[pl.ds(_L, _L)] = w1b
        w12[pl.ds(2 * _L, _L)] = w2a
        w12[pl.ds(3 * _L, _L)] = w2b
        i12[pl.ds(0, _L)] = p1a
        i12[pl.ds(_L, _L)] = p1b
        i12[pl.ds(2 * _L, _L)] = p2a
        i12[pl.ds(3 * _L, _L)] = p2b

        def cchunk(jj):
            ai = atv[pl.ds(_L * jj, _L)]
            bi = btv[pl.ds(_L * jj, _L)]
            pv = pvv[pl.ds(_L * jj, _L)]
            ga = plsc.load_gather(w12, [ai])
            gb = plsc.load_gather(w12, [bi])
            ia = plsc.load_gather(i12, [ai])
            ib = plsc.load_gather(i12, [bi])
            ck = jnp.maximum(ga + gb, 0.0) + pv
            cv = ia * _PKM + ib
            return _sortkv(ck, cv)

        r1, r2, v1, v2 = negv, negv, bigv, bigv
        for j in range(_NCAND // (2 * _L)):
            c1k, c1v = cchunk(2 * j)
            c2k, c2v = cchunk(2 * j + 1)
            t1k, t1v, t2k, t2v = _merge16_full(c1k, c1v, c2k, c2v)
            r1, r2, v1, v2 = _merge32_top(r1, v1, r2, v2,
                                          t1k, t1v, t2k, t2v)

        out_off = r_glob * _K
        wout[pl.ds(out_off, _L)] = r1
        wout[pl.ds(out_off + _L, _L)] = r2
        iout[pl.ds(out_off, _L)] = v1
        iout[pl.ds(out_off + _L, _L)] = v2

    def batch_body(b, _):
        pltpu.sync_copy(
            h_hbm.at[pl.ds((row0 + b * _RB) * (2 * _PAD), _RB * 2 * _PAD)],
            hbuf)
        pltpu.sync_copy(
            t_hbm.at[pl.ds((row0 + b * _RB) * 2, _RB * 2)],
            tbuf.at[pl.ds(0, _RB * 2)])

        def rloop(r, _2):
            row_body(b * _RB + r)
            return 0

        lax.fori_loop(0, _RB, rloop, 0)
        return 0

    lax.fori_loop(0, nbatch, batch_body, 0)
    pltpu.sync_copy(wout, ow_hbm.at[pl.ds(row0 * _K, rpw * _K)])
    pltpu.sync_copy(iout, oi_hbm.at[pl.ds(row0 * _K, rpw * _K)])
  return _sc_body


def _sc_topk(h_flat, t_flat, atab, btab, padv, ntok):
    rpw = ntok // _NW
    nbatch = rpw // _RB
    mesh = plsc.VectorSubcoreMesh(core_axis_name="c", subcore_axis_name="s",
                                  num_cores=_NC, num_subcores=_NS)
    f = pl.kernel(
        _make_sc_body(rpw, nbatch),
        out_type=(
            jax.ShapeDtypeStruct((ntok * _K,), jnp.float32),
            jax.ShapeDtypeStruct((ntok * _K,), jnp.int32),
        ),
        mesh=mesh,
        compiler_params=pltpu.CompilerParams(needs_layout_passes=False),
        scratch_types=[
            pltpu.VMEM((_RB * 2 * _PAD,), jnp.float32),
            pltpu.VMEM((_RB * 2 + _L,), jnp.float32),
            pltpu.VMEM((_PAD + 2 * _L,), jnp.float32),
            pltpu.VMEM((_PAD + 2 * _L,), jnp.int32),
            pltpu.VMEM((_PAD + 2 * _L,), jnp.float32),
            pltpu.VMEM((_PAD + 2 * _L,), jnp.int32),
            pltpu.VMEM((4 * _L,), jnp.float32),
            pltpu.VMEM((4 * _L,), jnp.int32),
            pltpu.VMEM((_NCAND,), jnp.int32),
            pltpu.VMEM((_NCAND,), jnp.int32),
            pltpu.VMEM((_NCAND,), jnp.float32),
            pltpu.VMEM((rpw * _K,), jnp.float32),
            pltpu.VMEM((rpw * _K,), jnp.int32),
        ],
    )
    return f(h_flat, t_flat, atab, btab, padv)


def _sc_tables():
    import numpy as np
    at = np.zeros((_NCAND,), np.int32)
    bt = np.zeros((_NCAND,), np.int32)
    pv = np.full((_NCAND,), _DEAD, np.float32)
    for j, (a, b) in enumerate(_AB):
        at[j] = a
        bt[j] = b + _K
        pv[j] = 0.0
    return jnp.asarray(at), jnp.asarray(bt), jnp.asarray(pv)


def kernel(x, W, b_lin, bias, k):
    del bias
    npad = _PAD - _PKM
    zrows = jnp.zeros((npad, _D_IN), W.dtype)
    wpt = jnp.concatenate([W[:_PKM], zrows, W[_PKM:], zrows], axis=0).T
    negs = jnp.full((npad,), _NEG_PAD, jnp.float32)
    bp = jnp.concatenate(
        [b_lin[:_PKM], negs, b_lin[_PKM:], negs]).reshape(1, 2 * _PAD)
    atab, btab, padv = _sc_tables()
    ws, is_ = [], []
    nchunk = _N_TOK // _NSPLIT
    for c in range(_NSPLIT):
        xc = x[c * nchunk:(c + 1) * nchunk]
        h, tt = _matmul(xc, wpt, bp)
        w_flat, i_flat = _sc_topk(h.reshape(-1), tt.reshape(-1),
                                  atab, btab, padv, nchunk)
        ws.append(w_flat.reshape(nchunk, _K))
        is_.append(i_flat.reshape(nchunk, _K))
    w = jnp.concatenate(ws, axis=0)
    i = jnp.concatenate(is_, axis=0)
    keep = jnp.asarray(k) == _K
    w = jnp.where(keep, w, jnp.zeros_like(w))
    i = jnp.where(keep, i, jnp.zeros_like(i))
    return w, i


# speedup vs baseline: 1.1849x; 1.0829x over previous
import functools

import jax
import jax.numpy as jnp
from jax import lax
from jax.experimental import pallas as pl
from jax.experimental.pallas import tpu as pltpu
from jax.experimental.pallas import tpu_sc as plsc

_D_IN = 2048
_PKM = 1000
_PAD = 1024
_K = 32
_N_TOK = 8192
_BLK = 256
_NEG_PAD = -1e30
_DEAD = -3e38

_NC = 2
_NS = 16
_L = 16
_NW = _NC * _NS
_RPW = _N_TOK // _NW
_RB = 16
_NBATCH = _RPW // _RB

_AB = [(a, b) for a in range(_K) for b in range(_K) if (a + 1) * (b + 1) <= _K]
_NCAND = 128
_NSPLIT = 1


def _mm_body(x_ref, w_ref, b_ref, h_ref, t_ref):
    h = jnp.dot(x_ref[...], w_ref[...], preferred_element_type=jnp.float32)
    h = h + b_ref[...]
    h_ref[...] = h

    def thresh(v):
        w = _PAD
        while w > _K:
            w //= 2
            v = jnp.maximum(v[:, :w], v[:, w:])
        return jnp.min(v, axis=1, keepdims=True)

    t_ref[...] = jnp.concatenate(
        [thresh(h[:, :_PAD]), thresh(h[:, _PAD:])], axis=1)


def _matmul(x, wpt, bp):
    ntok = x.shape[0]
    fixed = lambda i: (0, 0)
    return pl.pallas_call(
        _mm_body,
        grid=(ntok // _BLK,),
        in_specs=[
            pl.BlockSpec((_BLK, _D_IN), lambda i: (i, 0)),
            pl.BlockSpec((_D_IN, 2 * _PAD), fixed),
            pl.BlockSpec((1, 2 * _PAD), fixed),
        ],
        out_specs=[
            pl.BlockSpec((_BLK, 2 * _PAD), lambda i: (i, 0)),
            pl.BlockSpec((_BLK, 2), lambda i: (i, 0)),
        ],
        out_shape=[
            jax.ShapeDtypeStruct((ntok, 2 * _PAD), jnp.float32),
            jax.ShapeDtypeStruct((ntok, 2), jnp.float32),
        ],
        compiler_params=pltpu.CompilerParams(
            dimension_semantics=("parallel",),
        ),
    )(x, wpt, bp)


def _sortkv(keys, vals):
    return plsc.sort_key_val(keys, vals, descending=True)


def _minmax_kv(ak, av, bk, bv):
    m = ak >= bk
    hk = jnp.where(m, ak, bk)
    hv = jnp.where(m, av, bv)
    lk = jnp.where(m, bk, ak)
    lv = jnp.where(m, bv, av)
    return hk, hv, lk, lv


def _merge16_full(c1k, c1v, c2k, c2v):
    r2k = lax.rev(c2k, (0,))
    r2v = lax.rev(c2v, (0,))
    hk, hv, lk, lv = _minmax_kv(c1k, c1v, r2k, r2v)
    t1k, t1v = _sortkv(hk, hv)
    t2k, t2v = _sortkv(lk, lv)
    return t1k, t1v, t2k, t2v


def _merge32_top(r1k, r1v, r2k, r2v, t1k, t1v, t2k, t2v):
    x1k, x1v, _, _ = _minmax_kv(r1k, r1v, lax.rev(t2k, (0,)),
                                lax.rev(t2v, (0,)))
    x2k, x2v, _, _ = _minmax_kv(r2k, r2v, lax.rev(t1k, (0,)),
                                lax.rev(t1v, (0,)))
    y1k, y1v, y2k, y2v = _minmax_kv(x1k, x1v, x2k, x2v)
    r1k, r1v = _sortkv(y1k, y1v)
    r2k, r2v = _sortkv(y2k, y2v)
    return r1k, r2k, r1v, r2v


def _make_sc_body(rpw, nbatch):
  def _sc_body(h_hbm, t_hbm, at_hbm, bt_hbm, pv_hbm, ow_hbm, oi_hbm,
               hbuf, tbuf, sval, sidx, sval2, sidx2, w12, i12, atv, btv, pvv,
               wout, iout):
    wid = lax.axis_index("s") * _NC + lax.axis_index("c")
    row0 = wid * rpw
    pltpu.sync_copy(at_hbm, atv)
    pltpu.sync_copy(bt_hbm, btv)
    pltpu.sync_copy(pv_hbm, pvv)
    iota = lax.broadcasted_iota(jnp.int32, (_L,), 0)
    negv = jnp.full((_L,), _DEAD, jnp.float32)
    bigv = jnp.full((_L,), 1 << 30, jnp.int32)

    def compact_both(rl, t1, t2):
        def comp(j, c):
            oa, ob = c
            basea = 2 * _L * j
            baseb = basea + _PAD
            va1 = hbuf[rl, pl.ds(basea, _L)]
            va2 = hbuf[rl, pl.ds(basea + _L, _L)]
            vb1 = hbuf[rl, pl.ds(baseb, _L)]
            vb2 = hbuf[rl, pl.ds(baseb + _L, _L)]
            ma1 = va1 >= t1
            ma2 = va2 >= t1
            mb1 = vb1 >= t2
            mb2 = vb2 >= t2
            ix = iota + 2 * _L * j
            plsc.store_compressed(sval.at[pl.ds(oa, _L)], va1, mask=ma1)
            plsc.store_compressed(sidx.at[pl.ds(oa, _L)], ix, mask=ma1)
            plsc.store_compressed(sval2.at[pl.ds(ob, _L)], vb1, mask=mb1)
            plsc.store_compressed(sidx2.at[pl.ds(ob, _L)], ix, mask=mb1)
            oa1 = oa + plsc.all_reduce_population_count(ma1)[0]
            ob1 = ob + plsc.all_reduce_population_count(mb1)[0]
            plsc.store_compressed(sval.at[pl.ds(oa1, _L)], va2, mask=ma2)
            plsc.store_compressed(sidx.at[pl.ds(oa1, _L)], ix + _L, mask=ma2)
            plsc.store_compressed(sval2.at[pl.ds(ob1, _L)], vb2, mask=mb2)
            plsc.store_compressed(sidx2.at[pl.ds(ob1, _L)], ix + _L,
                                  mask=mb2)
            return (oa1 + plsc.all_reduce_population_count(ma2)[0],
                    ob1 + plsc.all_reduce_population_count(mb2)[0])

        oa, ob = lax.fori_loop(0, _PAD // (2 * _L), comp,
                               (jnp.int32(0), jnp.int32(0)))
        sval[pl.ds(oa, _L)] = negv
        sidx[pl.ds(oa, _L)] = bigv
        sval[pl.ds(oa + _L, _L)] = negv
        sidx[pl.ds(oa + _L, _L)] = bigv
        sval2[pl.ds(ob, _L)] = negv
        sidx2[pl.ds(ob, _L)] = bigv
        sval2[pl.ds(ob + _L, _L)] = negv
        sidx2[pl.ds(ob + _L, _L)] = bigv
        return oa, ob

    def merge_surv(vref, iref, o):
        npair = (o + 2 * _L - 1) // (2 * _L)

        def mstep(j, c):
            r1k, r2k, r1v, r2v = c
            b = 2 * _L * j
            c1k, c1v = _sortkv(vref[pl.ds(b, _L)], iref[pl.ds(b, _L)])
            c2k, c2v = _sortkv(vref[pl.ds(b + _L, _L)],
                               iref[pl.ds(b + _L, _L)])
            t1k, t1v, t2k, t2v = _merge16_full(c1k, c1v, c2k, c2v)
            return _merge32_top(r1k, r1v, r2k, r2v, t1k, t1v, t2k, t2v)

        return lax.fori_loop(0, npair, mstep, (negv, negv, bigv, bigv))

    def row_body(r_glob):
        rl = r_glob % _RB
        tv = tbuf[pl.ds(2 * rl, _L)]
        oa, ob = compact_both(rl, tv[0], tv[1])
        w1a, w1b, p1a, p1b = merge_surv(sval, sidx, oa)
        w2a, w2b, p2a, p2b = merge_surv(sval2, sidx2, ob)
        w12[pl.ds(0, _L)] = w1a
        w12[pl.ds(_L, _L)] = w1b
        w12[pl.ds(2 * _L, _L)] = w2a
        w12[pl.ds(3 * _L, _L)] = w2b
        i12[pl.ds(0, _L)] = p1a
        i12[pl.ds(_L, _L)] = p1b
        i12[pl.ds(2 * _L, _L)] = p2a
        i12[pl.ds(3 * _L, _L)] = p2b

        def cchunk(jj):
            ai = atv[pl.ds(_L * jj, _L)]
            bi = btv[pl.ds(_L * jj, _L)]
            pv = pvv[pl.ds(_L * jj, _L)]
            ga = plsc.load_gather(w12, [ai])
            gb = plsc.load_gather(w12, [bi])
            ia = plsc.load_gather(i12, [ai])
            ib = plsc.load_gather(i12, [bi])
            ck = jnp.maximum(ga + gb, 0.0) + pv
            cv = ia * _PKM + ib
            return _sortkv(ck, cv)

        r1, r2, v1, v2 = negv, negv, bigv, bigv
        for j in range(_NCAND // (2 * _L)):
            c1k, c1v = cchunk(2 * j)
            c2k, c2v = cchunk(2 * j + 1)
            t1k, t1v, t2k, t2v = _merge16_full(c1k, c1v, c2k, c2v)
            r1, r2, v1, v2 = _merge32_top(r1, v1, r2, v2,
                                          t1k, t1v, t2k, t2v)

        out_off = r_glob * _K
        wout[pl.ds(out_off, _L)] = r1
        wout[pl.ds(out_off + _L, _L)] = r2
        iout[pl.ds(out_off, _L)] = v1
        iout[pl.ds(out_off + _L, _L)] = v2

    def batch_body(b, _):
        pltpu.sync_copy(h_hbm.at[pl.ds(row0 + b * _RB, _RB), :], hbuf)
        pltpu.sync_copy(
            t_hbm.at[pl.ds((row0 + b * _RB) * 2, _RB * 2)],
            tbuf.at[pl.ds(0, _RB * 2)])

        def rloop(r, _2):
            row_body(b * _RB + r)
            return 0

        lax.fori_loop(0, _RB, rloop, 0)
        return 0

    lax.fori_loop(0, nbatch, batch_body, 0)
    pltpu.sync_copy(wout, ow_hbm.at[pl.ds(row0 * _K, rpw * _K)])
    pltpu.sync_copy(iout, oi_hbm.at[pl.ds(row0 * _K, rpw * _K)])
  return _sc_body


def _sc_topk(h2d, t_flat, atab, btab, padv, ntok):
    rpw = ntok // _NW
    nbatch = rpw // _RB
    mesh = plsc.VectorSubcoreMesh(core_axis_name="c", subcore_axis_name="s",
                                  num_cores=_NC, num_subcores=_NS)
    f = pl.kernel(
        _make_sc_body(rpw, nbatch),
        out_type=(
            jax.ShapeDtypeStruct((ntok * _K,), jnp.float32),
            jax.ShapeDtypeStruct((ntok * _K,), jnp.int32),
        ),
        mesh=mesh,
        compiler_params=pltpu.CompilerParams(needs_layout_passes=False),
        scratch_types=[
            pltpu.VMEM((_RB, 2 * _PAD), jnp.float32),
            pltpu.VMEM((_RB * 2 + _L,), jnp.float32),
            pltpu.VMEM((_PAD + 2 * _L,), jnp.float32),
            pltpu.VMEM((_PAD + 2 * _L,), jnp.int32),
            pltpu.VMEM((_PAD + 2 * _L,), jnp.float32),
            pltpu.VMEM((_PAD + 2 * _L,), jnp.int32),
            pltpu.VMEM((4 * _L,), jnp.float32),
            pltpu.VMEM((4 * _L,), jnp.int32),
            pltpu.VMEM((_NCAND,), jnp.int32),
            pltpu.VMEM((_NCAND,), jnp.int32),
            pltpu.VMEM((_NCAND,), jnp.float32),
            pltpu.VMEM((rpw * _K,), jnp.float32),
            pltpu.VMEM((rpw * _K,), jnp.int32),
        ],
    )
    return f(h2d, t_flat, atab, btab, padv)


def _sc_tables():
    import numpy as np
    at = np.zeros((_NCAND,), np.int32)
    bt = np.zeros((_NCAND,), np.int32)
    pv = np.full((_NCAND,), _DEAD, np.float32)
    for j, (a, b) in enumerate(_AB):
        at[j] = a
        bt[j] = b + _K
        pv[j] = 0.0
    return jnp.asarray(at), jnp.asarray(bt), jnp.asarray(pv)


def kernel(x, W, b_lin, bias, k):
    del bias
    npad = _PAD - _PKM
    zrows = jnp.zeros((npad, _D_IN), W.dtype)
    wpt = jnp.concatenate([W[:_PKM], zrows, W[_PKM:], zrows], axis=0).T
    negs = jnp.full((npad,), _NEG_PAD, jnp.float32)
    bp = jnp.concatenate(
        [b_lin[:_PKM], negs, b_lin[_PKM:], negs]).reshape(1, 2 * _PAD)
    atab, btab, padv = _sc_tables()
    ws, is_ = [], []
    nchunk = _N_TOK // _NSPLIT
    for c in range(_NSPLIT):
        xc = x[c * nchunk:(c + 1) * nchunk]
        h, tt = _matmul(xc, wpt, bp)
        w_flat, i_flat = _sc_topk(h, tt.reshape(-1),
                                  atab, btab, padv, nchunk)
        ws.append(w_flat.reshape(nchunk, _K))
        is_.append(i_flat.reshape(nchunk, _K))
    w = jnp.concatenate(ws, axis=0)
    i = jnp.concatenate(is_, axis=0)
    keep = jnp.asarray(k) == _K
    w = jnp.where(keep, w, jnp.zeros_like(w))
    i = jnp.where(keep, i, jnp.zeros_like(i))
    return w, i


# speedup vs baseline: 1.2033x; 1.0155x over previous
import functools

import jax
import jax.numpy as jnp
from jax import lax
from jax.experimental import pallas as pl
from jax.experimental.pallas import tpu as pltpu
from jax.experimental.pallas import tpu_sc as plsc

_D_IN = 2048
_PKM = 1000
_PAD = 1024
_K = 32
_N_TOK = 8192
_BLK = 256
_NEG_PAD = -1e30
_DEAD = -3e38

_NC = 2
_NS = 16
_L = 16
_NW = _NC * _NS
_RPW = _N_TOK // _NW
_RB = 32
_NBATCH = _RPW // _RB

_AB = [(a, b) for a in range(_K) for b in range(_K) if (a + 1) * (b + 1) <= _K]
_NCAND = 128
_NSPLIT = 1


def _mm_body(x_ref, w_ref, b_ref, h_ref, t_ref):
    h = jnp.dot(x_ref[...], w_ref[...], preferred_element_type=jnp.float32)
    h = h + b_ref[...]
    h_ref[...] = h

    def thresh(v):
        w = _PAD
        while w > _K:
            w //= 2
            v = jnp.maximum(v[:, :w], v[:, w:])
        return jnp.min(v, axis=1, keepdims=True)

    t_ref[...] = jnp.concatenate(
        [thresh(h[:, :_PAD]), thresh(h[:, _PAD:])], axis=1)


def _matmul(x, wpt, bp):
    ntok = x.shape[0]
    fixed = lambda i: (0, 0)
    return pl.pallas_call(
        _mm_body,
        grid=(ntok // _BLK,),
        in_specs=[
            pl.BlockSpec((_BLK, _D_IN), lambda i: (i, 0)),
            pl.BlockSpec((_D_IN, 2 * _PAD), fixed),
            pl.BlockSpec((1, 2 * _PAD), fixed),
        ],
        out_specs=[
            pl.BlockSpec((_BLK, 2 * _PAD), lambda i: (i, 0)),
            pl.BlockSpec((_BLK, 2), lambda i: (i, 0)),
        ],
        out_shape=[
            jax.ShapeDtypeStruct((ntok, 2 * _PAD), jnp.float32),
            jax.ShapeDtypeStruct((ntok, 2), jnp.float32),
        ],
        compiler_params=pltpu.CompilerParams(
            dimension_semantics=("parallel",),
        ),
    )(x, wpt, bp)


def _sortkv(keys, vals):
    return plsc.sort_key_val(keys, vals, descending=True)


def _minmax_kv(ak, av, bk, bv):
    m = ak >= bk
    hk = jnp.where(m, ak, bk)
    hv = jnp.where(m, av, bv)
    lk = jnp.where(m, bk, ak)
    lv = jnp.where(m, bv, av)
    return hk, hv, lk, lv


def _merge16_full(c1k, c1v, c2k, c2v):
    r2k = lax.rev(c2k, (0,))
    r2v = lax.rev(c2v, (0,))
    hk, hv, lk, lv = _minmax_kv(c1k, c1v, r2k, r2v)
    t1k, t1v = _sortkv(hk, hv)
    t2k, t2v = _sortkv(lk, lv)
    return t1k, t1v, t2k, t2v


def _merge32_top(r1k, r1v, r2k, r2v, t1k, t1v, t2k, t2v):
    x1k, x1v, _, _ = _minmax_kv(r1k, r1v, lax.rev(t2k, (0,)),
                                lax.rev(t2v, (0,)))
    x2k, x2v, _, _ = _minmax_kv(r2k, r2v, lax.rev(t1k, (0,)),
                                lax.rev(t1v, (0,)))
    y1k, y1v, y2k, y2v = _minmax_kv(x1k, x1v, x2k, x2v)
    r1k, r1v = _sortkv(y1k, y1v)
    r2k, r2v = _sortkv(y2k, y2v)
    return r1k, r2k, r1v, r2v


def _make_sc_body(rpw, nbatch):
  def _sc_body(h_hbm, t_hbm, at_hbm, bt_hbm, pv_hbm, ow_hbm, oi_hbm,
               hbuf, tbuf, sval, sidx, sval2, sidx2, w12, i12, atv, btv, pvv,
               wout, iout):
    wid = lax.axis_index("s") * _NC + lax.axis_index("c")
    row0 = wid * rpw
    pltpu.sync_copy(at_hbm, atv)
    pltpu.sync_copy(bt_hbm, btv)
    pltpu.sync_copy(pv_hbm, pvv)
    iota = lax.broadcasted_iota(jnp.int32, (_L,), 0)
    negv = jnp.full((_L,), _DEAD, jnp.float32)
    bigv = jnp.full((_L,), 1 << 30, jnp.int32)

    def compact_both(rl, t1, t2):
        def comp(j, c):
            oa, ob = c
            basea = 2 * _L * j
            baseb = basea + _PAD
            va1 = hbuf[rl, pl.ds(basea, _L)]
            va2 = hbuf[rl, pl.ds(basea + _L, _L)]
            vb1 = hbuf[rl, pl.ds(baseb, _L)]
            vb2 = hbuf[rl, pl.ds(baseb + _L, _L)]
            ma1 = va1 >= t1
            ma2 = va2 >= t1
            mb1 = vb1 >= t2
            mb2 = vb2 >= t2
            ix = iota + 2 * _L * j
            plsc.store_compressed(sval.at[pl.ds(oa, _L)], va1, mask=ma1)
            plsc.store_compressed(sidx.at[pl.ds(oa, _L)], ix, mask=ma1)
            plsc.store_compressed(sval2.at[pl.ds(ob, _L)], vb1, mask=mb1)
            plsc.store_compressed(sidx2.at[pl.ds(ob, _L)], ix, mask=mb1)
            oa1 = oa + plsc.all_reduce_population_count(ma1)[0]
            ob1 = ob + plsc.all_reduce_population_count(mb1)[0]
            plsc.store_compressed(sval.at[pl.ds(oa1, _L)], va2, mask=ma2)
            plsc.store_compressed(sidx.at[pl.ds(oa1, _L)], ix + _L, mask=ma2)
            plsc.store_compressed(sval2.at[pl.ds(ob1, _L)], vb2, mask=mb2)
            plsc.store_compressed(sidx2.at[pl.ds(ob1, _L)], ix + _L,
                                  mask=mb2)
            return (oa1 + plsc.all_reduce_population_count(ma2)[0],
                    ob1 + plsc.all_reduce_population_count(mb2)[0])

        oa, ob = lax.fori_loop(0, _PAD // (2 * _L), comp,
                               (jnp.int32(0), jnp.int32(0)))
        sval[pl.ds(oa, _L)] = negv
        sidx[pl.ds(oa, _L)] = bigv
        sval[pl.ds(oa + _L, _L)] = negv
        sidx[pl.ds(oa + _L, _L)] = bigv
        sval2[pl.ds(ob, _L)] = negv
        sidx2[pl.ds(ob, _L)] = bigv
        sval2[pl.ds(ob + _L, _L)] = negv
        sidx2[pl.ds(ob + _L, _L)] = bigv
        return oa, ob

    def merge_surv(vref, iref, o):
        npair = (o + 2 * _L - 1) // (2 * _L)

        def mstep(j, c):
            r1k, r2k, r1v, r2v = c
            b = 2 * _L * j
            c1k, c1v = _sortkv(vref[pl.ds(b, _L)], iref[pl.ds(b, _L)])
            c2k, c2v = _sortkv(vref[pl.ds(b + _L, _L)],
                               iref[pl.ds(b + _L, _L)])
            t1k, t1v, t2k, t2v = _merge16_full(c1k, c1v, c2k, c2v)
            return _merge32_top(r1k, r1v, r2k, r2v, t1k, t1v, t2k, t2v)

        return lax.fori_loop(0, npair, mstep, (negv, negv, bigv, bigv))

    def row_body(r_glob):
        rl = r_glob % _RB
        tv = tbuf[pl.ds(2 * rl, _L)]
        oa, ob = compact_both(rl, tv[0], tv[1])
        w1a, w1b, p1a, p1b = merge_surv(sval, sidx, oa)
        w2a, w2b, p2a, p2b = merge_surv(sval2, sidx2, ob)
        w12[pl.ds(0, _L)] = w1a
        w12[pl.ds(_L, _L)] = w1b
        w12[pl.ds(2 * _L, _L)] = w2a
        w12[pl.ds(3 * _L, _L)] = w2b
        i12[pl.ds(0, _L)] = p1a
        i12[pl.ds(_L, _L)] = p1b
        i12[pl.ds(2 * _L, _L)] = p2a
        i12[pl.ds(3 * _L, _L)] = p2b

        def cchunk(jj):
            ai = atv[pl.ds(_L * jj, _L)]
            bi = btv[pl.ds(_L * jj, _L)]
            pv = pvv[pl.ds(_L * jj, _L)]
            ga = plsc.load_gather(w12, [ai])
            gb = plsc.load_gather(w12, [bi])
            ia = plsc.load_gather(i12, [ai])
            ib = plsc.load_gather(i12, [bi])
            ck = jnp.maximum(ga + gb, 0.0) + pv
            cv = ia * _PKM + ib
            return _sortkv(ck, cv)

        r1, r2, v1, v2 = negv, negv, bigv, bigv
        for j in range(_NCAND // (2 * _L)):
            c1k, c1v = cchunk(2 * j)
            c2k, c2v = cchunk(2 * j + 1)
            t1k, t1v, t2k, t2v = _merge16_full(c1k, c1v, c2k, c2v)
            r1, r2, v1, v2 = _merge32_top(r1, v1, r2, v2,
                                          t1k, t1v, t2k, t2v)

        out_off = r_glob * _K
        wout[pl.ds(out_off, _L)] = r1
        wout[pl.ds(out_off + _L, _L)] = r2
        iout[pl.ds(out_off, _L)] = v1
        iout[pl.ds(out_off + _L, _L)] = v2

    def batch_body(b, _):
        pltpu.sync_copy(h_hbm.at[pl.ds(row0 + b * _RB, _RB), :], hbuf)
        pltpu.sync_copy(
            t_hbm.at[pl.ds((row0 + b * _RB) * 2, _RB * 2)],
            tbuf.at[pl.ds(0, _RB * 2)])

        def rloop(r, _2):
            row_body(b * _RB + r)
            return 0

        lax.fori_loop(0, _RB, rloop, 0)
        return 0

    lax.fori_loop(0, nbatch, batch_body, 0)
    pltpu.sync_copy(wout, ow_hbm.at[pl.ds(row0 * _K, rpw * _K)])
    pltpu.sync_copy(iout, oi_hbm.at[pl.ds(row0 * _K, rpw * _K)])
  return _sc_body


def _sc_topk(h2d, t_flat, atab, btab, padv, ntok):
    rpw = ntok // _NW
    nbatch = rpw // _RB
    mesh = plsc.VectorSubcoreMesh(core_axis_name="c", subcore_axis_name="s",
                                  num_cores=_NC, num_subcores=_NS)
    f = pl.kernel(
        _make_sc_body(rpw, nbatch),
        out_type=(
            jax.ShapeDtypeStruct((ntok * _K,), jnp.float32),
            jax.ShapeDtypeStruct((ntok * _K,), jnp.int32),
        ),
        mesh=mesh,
        compiler_params=pltpu.CompilerParams(needs_layout_passes=False),
        scratch_types=[
            pltpu.VMEM((_RB, 2 * _PAD), jnp.float32),
            pltpu.VMEM((_RB * 2 + _L,), jnp.float32),
            pltpu.VMEM((_PAD + 2 * _L,), jnp.float32),
            pltpu.VMEM((_PAD + 2 * _L,), jnp.int32),
            pltpu.VMEM((_PAD + 2 * _L,), jnp.float32),
            pltpu.VMEM((_PAD + 2 * _L,), jnp.int32),
            pltpu.VMEM((4 * _L,), jnp.float32),
            pltpu.VMEM((4 * _L,), jnp.int32),
            pltpu.VMEM((_NCAND,), jnp.int32),
            pltpu.VMEM((_NCAND,), jnp.int32),
            pltpu.VMEM((_NCAND,), jnp.float32),
            pltpu.VMEM((rpw * _K,), jnp.float32),
            pltpu.VMEM((rpw * _K,), jnp.int32),
        ],
    )
    return f(h2d, t_flat, atab, btab, padv)


def _sc_tables():
    import numpy as np
    at = np.zeros((_NCAND,), np.int32)
    bt = np.zeros((_NCAND,), np.int32)
    pv = np.full((_NCAND,), _DEAD, np.float32)
    for j, (a, b) in enumerate(_AB):
        at[j] = a
        bt[j] = b + _K
        pv[j] = 0.0
    return jnp.asarray(at), jnp.asarray(bt), jnp.asarray(pv)


def kernel(x, W, b_lin, bias, k):
    del bias
    npad = _PAD - _PKM
    zrows = jnp.zeros((npad, _D_IN), W.dtype)
    wpt = jnp.concatenate([W[:_PKM], zrows, W[_PKM:], zrows], axis=0).T
    negs = jnp.full((npad,), _NEG_PAD, jnp.float32)
    bp = jnp.concatenate(
        [b_lin[:_PKM], negs, b_lin[_PKM:], negs]).reshape(1, 2 * _PAD)
    atab, btab, padv = _sc_tables()
    ws, is_ = [], []
    nchunk = _N_TOK // _NSPLIT
    for c in range(_NSPLIT):
        xc = x[c * nchunk:(c + 1) * nchunk]
        h, tt = _matmul(xc, wpt, bp)
        w_flat, i_flat = _sc_topk(h, tt.reshape(-1),
                                  atab, btab, padv, nchunk)
        ws.append(w_flat.reshape(nchunk, _K))
        is_.append(i_flat.reshape(nchunk, _K))
    w = jnp.concatenate(ws, axis=0)
    i = jnp.concatenate(is_, axis=0)
    keep = jnp.asarray(k) == _K
    w = jnp.where(keep, w, jnp.zeros_like(w))
    i = jnp.where(keep, i, jnp.zeros_like(i))
    return w, i


# speedup vs baseline: 1.2510x; 1.0397x over previous
import functools

import jax
import jax.numpy as jnp
from jax import lax
from jax.experimental import pallas as pl
from jax.experimental.pallas import tpu as pltpu
from jax.experimental.pallas import tpu_sc as plsc

_D_IN = 2048
_PKM = 1000
_PAD = 1024
_K = 32
_N_TOK = 8192
_BLK = 256
_NEG_PAD = -1e30
_DEAD = -3e38

_NC = 2
_NS = 16
_L = 16
_NW = _NC * _NS
_RPW = _N_TOK // _NW
_RB = 32
_NBATCH = _RPW // _RB

_AB = [(a, b) for a in range(_K) for b in range(_K) if (a + 1) * (b + 1) <= _K]
_NCAND = 128
_NSPLIT = 1


def _mm_body(x_ref, w_ref, b_ref, h_ref, t_ref):
    h = jnp.dot(x_ref[...], w_ref[...], preferred_element_type=jnp.float32)
    h = h + b_ref[...]
    h_ref[...] = h

    def thresh(v):
        w = _PAD
        while w > _K:
            w //= 2
            v = jnp.maximum(v[:, :w], v[:, w:])
        return jnp.min(v, axis=1, keepdims=True)

    t_ref[...] = jnp.concatenate(
        [thresh(h[:, :_PAD]), thresh(h[:, _PAD:])], axis=1)


def _matmul(x, wpt, bp):
    ntok = x.shape[0]
    fixed = lambda i: (0, 0)
    return pl.pallas_call(
        _mm_body,
        grid=(ntok // _BLK,),
        in_specs=[
            pl.BlockSpec((_BLK, _D_IN), lambda i: (i, 0)),
            pl.BlockSpec((_D_IN, 2 * _PAD), fixed),
            pl.BlockSpec((1, 2 * _PAD), fixed),
        ],
        out_specs=[
            pl.BlockSpec((_BLK, 2 * _PAD), lambda i: (i, 0)),
            pl.BlockSpec((_BLK, 2), lambda i: (i, 0)),
        ],
        out_shape=[
            jax.ShapeDtypeStruct((ntok, 2 * _PAD), jnp.float32),
            jax.ShapeDtypeStruct((ntok, 2), jnp.float32),
        ],
        compiler_params=pltpu.CompilerParams(
            dimension_semantics=("parallel",),
        ),
    )(x, wpt, bp)


def _sortkv(keys, vals):
    return plsc.sort_key_val(keys, vals, descending=True)


def _minmax_kv(ak, av, bk, bv):
    m = ak >= bk
    hk = jnp.where(m, ak, bk)
    hv = jnp.where(m, av, bv)
    lk = jnp.where(m, bk, ak)
    lv = jnp.where(m, bv, av)
    return hk, hv, lk, lv


def _merge16_full(c1k, c1v, c2k, c2v):
    r2k = lax.rev(c2k, (0,))
    r2v = lax.rev(c2v, (0,))
    hk, hv, lk, lv = _minmax_kv(c1k, c1v, r2k, r2v)
    t1k, t1v = _sortkv(hk, hv)
    t2k, t2v = _sortkv(lk, lv)
    return t1k, t1v, t2k, t2v


def _merge32_top(r1k, r1v, r2k, r2v, t1k, t1v, t2k, t2v):
    x1k, x1v, _, _ = _minmax_kv(r1k, r1v, lax.rev(t2k, (0,)),
                                lax.rev(t2v, (0,)))
    x2k, x2v, _, _ = _minmax_kv(r2k, r2v, lax.rev(t1k, (0,)),
                                lax.rev(t1v, (0,)))
    y1k, y1v, y2k, y2v = _minmax_kv(x1k, x1v, x2k, x2v)
    r1k, r1v = _sortkv(y1k, y1v)
    r2k, r2v = _sortkv(y2k, y2v)
    return r1k, r1v, r2k, r2v


def _make_sc_body(rpw, nbatch):
  def _sc_body(h_hbm, t_hbm, at_hbm, bt_hbm, pv_hbm, ow_hbm, oi_hbm,
               hbuf, tbuf, sval, sidx, sval2, sidx2, w12, i12, atv, btv, pvv,
               wout, iout):
    wid = lax.axis_index("s") * _NC + lax.axis_index("c")
    row0 = wid * rpw
    pltpu.sync_copy(at_hbm, atv)
    pltpu.sync_copy(bt_hbm, btv)
    pltpu.sync_copy(pv_hbm, pvv)
    iota = lax.broadcasted_iota(jnp.int32, (_L,), 0)
    negv = jnp.full((_L,), _DEAD, jnp.float32)
    bigv = jnp.full((_L,), 1 << 30, jnp.int32)

    def compact_both(rl, t1, t2):
        def comp(j, c):
            oa, ob = c
            basea = 2 * _L * j
            baseb = basea + _PAD
            va1 = hbuf[rl, pl.ds(basea, _L)]
            va2 = hbuf[rl, pl.ds(basea + _L, _L)]
            vb1 = hbuf[rl, pl.ds(baseb, _L)]
            vb2 = hbuf[rl, pl.ds(baseb + _L, _L)]
            ma1 = va1 >= t1
            ma2 = va2 >= t1
            mb1 = vb1 >= t2
            mb2 = vb2 >= t2
            ix = iota + 2 * _L * j
            plsc.store_compressed(sval.at[pl.ds(oa, _L)], va1, mask=ma1)
            plsc.store_compressed(sidx.at[pl.ds(oa, _L)], ix, mask=ma1)
            plsc.store_compressed(sval2.at[pl.ds(ob, _L)], vb1, mask=mb1)
            plsc.store_compressed(sidx2.at[pl.ds(ob, _L)], ix, mask=mb1)
            oa1 = oa + plsc.all_reduce_population_count(ma1)[0]
            ob1 = ob + plsc.all_reduce_population_count(mb1)[0]
            plsc.store_compressed(sval.at[pl.ds(oa1, _L)], va2, mask=ma2)
            plsc.store_compressed(sidx.at[pl.ds(oa1, _L)], ix + _L, mask=ma2)
            plsc.store_compressed(sval2.at[pl.ds(ob1, _L)], vb2, mask=mb2)
            plsc.store_compressed(sidx2.at[pl.ds(ob1, _L)], ix + _L,
                                  mask=mb2)
            return (oa1 + plsc.all_reduce_population_count(ma2)[0],
                    ob1 + plsc.all_reduce_population_count(mb2)[0])

        oa, ob = lax.fori_loop(0, _PAD // (2 * _L), comp,
                               (jnp.int32(0), jnp.int32(0)))
        sval[pl.ds(oa, _L)] = negv
        sidx[pl.ds(oa, _L)] = bigv
        sval[pl.ds(oa + _L, _L)] = negv
        sidx[pl.ds(oa + _L, _L)] = bigv
        sval2[pl.ds(ob, _L)] = negv
        sidx2[pl.ds(ob, _L)] = bigv
        sval2[pl.ds(ob + _L, _L)] = negv
        sidx2[pl.ds(ob + _L, _L)] = bigv
        return oa, ob

    def merge_both(oa, ob):
        npa = (oa + 2 * _L - 1) // (2 * _L)
        npb = (ob + 2 * _L - 1) // (2 * _L)
        npmax = jnp.maximum(npa, npb)

        def one(vref, iref, b, r):
            c1k, c1v = _sortkv(vref[pl.ds(b, _L)], iref[pl.ds(b, _L)])
            c2k, c2v = _sortkv(vref[pl.ds(b + _L, _L)],
                               iref[pl.ds(b + _L, _L)])
            t1k, t1v, t2k, t2v = _merge16_full(c1k, c1v, c2k, c2v)
            return _merge32_top(*r, t1k, t1v, t2k, t2v)

        def mstep(j, c):
            ra, rb = c[:4], c[4:]
            b = 2 * _L * j
            na = one(sval, sidx, b, ra)
            nb = one(sval2, sidx2, b, rb)
            ina = j < npa
            inb = j < npb
            ra = tuple(jnp.where(ina, n, o) for n, o in zip(na, ra))
            rb = tuple(jnp.where(inb, n, o) for n, o in zip(nb, rb))
            return ra + rb

        init = (negv, bigv, negv, bigv)
        return lax.fori_loop(0, npmax, mstep, init + init)

    def row_body(r_glob):
        rl = r_glob % _RB
        tv = tbuf[pl.ds(2 * rl, _L)]
        oa, ob = compact_both(rl, tv[0], tv[1])
        (w1a, p1a, w1b, p1b,
         w2a, p2a, w2b, p2b) = merge_both(oa, ob)
        w12[pl.ds(0, _L)] = w1a
        w12[pl.ds(_L, _L)] = w1b
        w12[pl.ds(2 * _L, _L)] = w2a
        w12[pl.ds(3 * _L, _L)] = w2b
        i12[pl.ds(0, _L)] = p1a
        i12[pl.ds(_L, _L)] = p1b
        i12[pl.ds(2 * _L, _L)] = p2a
        i12[pl.ds(3 * _L, _L)] = p2b

        def cchunk(jj):
            ai = atv[pl.ds(_L * jj, _L)]
            bi = btv[pl.ds(_L * jj, _L)]
            pv = pvv[pl.ds(_L * jj, _L)]
            ga = plsc.load_gather(w12, [ai])
            gb = plsc.load_gather(w12, [bi])
            ia = plsc.load_gather(i12, [ai])
            ib = plsc.load_gather(i12, [bi])
            ck = jnp.maximum(ga + gb, 0.0) + pv
            cv = ia * _PKM + ib
            return _sortkv(ck, cv)

        def cpair(j):
            c1k, c1v = cchunk(2 * j)
            c2k, c2v = cchunk(2 * j + 1)
            return _merge16_full(c1k, c1v, c2k, c2v)

        t0, t1_, t2_, t3 = cpair(0), cpair(1), cpair(2), cpair(3)
        r01 = _merge32_top(*t0, *t1_)
        r23 = _merge32_top(*t2_, *t3)
        r1, v1, r2, v2 = _merge32_top(*r01, *r23)

        out_off = r_glob * _K
        wout[pl.ds(out_off, _L)] = r1
        wout[pl.ds(out_off + _L, _L)] = r2
        iout[pl.ds(out_off, _L)] = v1
        iout[pl.ds(out_off + _L, _L)] = v2

    def batch_body(b, _):
        pltpu.sync_copy(h_hbm.at[pl.ds(row0 + b * _RB, _RB), :], hbuf)
        pltpu.sync_copy(
            t_hbm.at[pl.ds((row0 + b * _RB) * 2, _RB * 2)],
            tbuf.at[pl.ds(0, _RB * 2)])

        def rloop(r, _2):
            row_body(b * _RB + r)
            return 0

        lax.fori_loop(0, _RB, rloop, 0)
        return 0

    lax.fori_loop(0, nbatch, batch_body, 0)
    pltpu.sync_copy(wout, ow_hbm.at[pl.ds(row0 * _K, rpw * _K)])
    pltpu.sync_copy(iout, oi_hbm.at[pl.ds(row0 * _K, rpw * _K)])
  return _sc_body


def _sc_topk(h2d, t_flat, atab, btab, padv, ntok):
    rpw = ntok // _NW
    nbatch = rpw // _RB
    mesh = plsc.VectorSubcoreMesh(core_axis_name="c", subcore_axis_name="s",
                                  num_cores=_NC, num_subcores=_NS)
    f = pl.kernel(
        _make_sc_body(rpw, nbatch),
        out_type=(
            jax.ShapeDtypeStruct((ntok * _K,), jnp.float32),
            jax.ShapeDtypeStruct((ntok * _K,), jnp.int32),
        ),
        mesh=mesh,
        compiler_params=pltpu.CompilerParams(needs_layout_passes=False),
        scratch_types=[
            pltpu.VMEM((_RB, 2 * _PAD), jnp.float32),
            pltpu.VMEM((_RB * 2 + _L,), jnp.float32),
            pltpu.VMEM((_PAD + 2 * _L,), jnp.float32),
            pltpu.VMEM((_PAD + 2 * _L,), jnp.int32),
            pltpu.VMEM((_PAD + 2 * _L,), jnp.float32),
            pltpu.VMEM((_PAD + 2 * _L,), jnp.int32),
            pltpu.VMEM((4 * _L,), jnp.float32),
            pltpu.VMEM((4 * _L,), jnp.int32),
            pltpu.VMEM((_NCAND,), jnp.int32),
            pltpu.VMEM((_NCAND,), jnp.int32),
            pltpu.VMEM((_NCAND,), jnp.float32),
            pltpu.VMEM((rpw * _K,), jnp.float32),
            pltpu.VMEM((rpw * _K,), jnp.int32),
        ],
    )
    return f(h2d, t_flat, atab, btab, padv)


def _sc_tables():
    import numpy as np
    at = np.zeros((_NCAND,), np.int32)
    bt = np.zeros((_NCAND,), np.int32)
    pv = np.full((_NCAND,), _DEAD, np.float32)
    for j, (a, b) in enumerate(_AB):
        at[j] = a
        bt[j] = b + _K
        pv[j] = 0.0
    return jnp.asarray(at), jnp.asarray(bt), jnp.asarray(pv)


def kernel(x, W, b_lin, bias, k):
    del bias
    npad = _PAD - _PKM
    zrows = jnp.zeros((npad, _D_IN), W.dtype)
    wpt = jnp.concatenate([W[:_PKM], zrows, W[_PKM:], zrows], axis=0).T
    negs = jnp.full((npad,), _NEG_PAD, jnp.float32)
    bp = jnp.concatenate(
        [b_lin[:_PKM], negs, b_lin[_PKM:], negs]).reshape(1, 2 * _PAD)
    atab, btab, padv = _sc_tables()
    ws, is_ = [], []
    nchunk = _N_TOK // _NSPLIT
    for c in range(_NSPLIT):
        xc = x[c * nchunk:(c + 1) * nchunk]
        h, tt = _matmul(xc, wpt, bp)
        w_flat, i_flat = _sc_topk(h, tt.reshape(-1),
                                  atab, btab, padv, nchunk)
        ws.append(w_flat.reshape(nchunk, _K))
        is_.append(i_flat.reshape(nchunk, _K))
    w = jnp.concatenate(ws, axis=0)
    i = jnp.concatenate(is_, axis=0)
    keep = jnp.asarray(k) == _K
    w = jnp.where(keep, w, jnp.zeros_like(w))
    i = jnp.where(keep, i, jnp.zeros_like(i))
    return w, i


# speedup vs baseline: 1.3189x; 1.0542x over previous
import functools

import jax
import jax.numpy as jnp
from jax import lax
from jax.experimental import pallas as pl
from jax.experimental.pallas import tpu as pltpu
from jax.experimental.pallas import tpu_sc as plsc

_D_IN = 2048
_PKM = 1000
_PAD = 1024
_K = 32
_N_TOK = 8192
_BLK = 256
_NEG_PAD = -1e30
_DEAD = -3e38

_NC = 2
_NS = 16
_L = 16
_NW = _NC * _NS
_RPW = _N_TOK // _NW
_RB = 32
_NBATCH = _RPW // _RB

_AB = [(a, b) for a in range(_K) for b in range(_K) if (a + 1) * (b + 1) <= _K]
_NCAND = 128
_NSPLIT = 1


def _mm_body(x_ref, w_ref, b_ref, h_ref, t_ref):
    h = jnp.dot(x_ref[...], w_ref[...], preferred_element_type=jnp.float32)
    h = h + b_ref[...]
    h_ref[...] = h

    def thresh(v):
        w = _PAD
        while w > _K:
            w //= 2
            v = jnp.maximum(v[:, :w], v[:, w:])
        return jnp.min(v, axis=1, keepdims=True)

    t_ref[...] = jnp.concatenate(
        [thresh(h[:, :_PAD]), thresh(h[:, _PAD:])], axis=1)


def _matmul(x, wpt, bp):
    ntok = x.shape[0]
    fixed = lambda i: (0, 0)
    return pl.pallas_call(
        _mm_body,
        grid=(ntok // _BLK,),
        in_specs=[
            pl.BlockSpec((_BLK, _D_IN), lambda i: (i, 0)),
            pl.BlockSpec((_D_IN, 2 * _PAD), fixed),
            pl.BlockSpec((1, 2 * _PAD), fixed),
        ],
        out_specs=[
            pl.BlockSpec((_BLK, 2 * _PAD), lambda i: (i, 0)),
            pl.BlockSpec((_BLK, 2), lambda i: (i, 0)),
        ],
        out_shape=[
            jax.ShapeDtypeStruct((ntok, 2 * _PAD), jnp.float32),
            jax.ShapeDtypeStruct((ntok, 2), jnp.float32),
        ],
        compiler_params=pltpu.CompilerParams(
            dimension_semantics=("parallel",),
        ),
    )(x, wpt, bp)


def _sortkv(keys, vals):
    return plsc.sort_key_val(keys, vals, descending=True)


def _minmax_kv(ak, av, bk, bv):
    m = ak >= bk
    hk = jnp.where(m, ak, bk)
    hv = jnp.where(m, av, bv)
    lk = jnp.where(m, bk, ak)
    lv = jnp.where(m, bv, av)
    return hk, hv, lk, lv


def _merge16_full(c1k, c1v, c2k, c2v):
    r2k = lax.rev(c2k, (0,))
    r2v = lax.rev(c2v, (0,))
    hk, hv, lk, lv = _minmax_kv(c1k, c1v, r2k, r2v)
    t1k, t1v = _sortkv(hk, hv)
    t2k, t2v = _sortkv(lk, lv)
    return t1k, t1v, t2k, t2v


def _merge32_top(r1k, r1v, r2k, r2v, t1k, t1v, t2k, t2v):
    x1k, x1v, _, _ = _minmax_kv(r1k, r1v, lax.rev(t2k, (0,)),
                                lax.rev(t2v, (0,)))
    x2k, x2v, _, _ = _minmax_kv(r2k, r2v, lax.rev(t1k, (0,)),
                                lax.rev(t1v, (0,)))
    y1k, y1v, y2k, y2v = _minmax_kv(x1k, x1v, x2k, x2v)
    r1k, r1v = _sortkv(y1k, y1v)
    r2k, r2v = _sortkv(y2k, y2v)
    return r1k, r1v, r2k, r2v


def _make_sc_body(rpw, nbatch):
  def _sc_body(h_hbm, t_hbm, at_hbm, bt_hbm, pv_hbm, ow_hbm, oi_hbm,
               hbuf, tbuf, sval, sidx, sval2, sidx2, w12, i12, atv, btv, pvv,
               wout, iout):
    wid = lax.axis_index("s") * _NC + lax.axis_index("c")
    row0 = wid * rpw
    pltpu.sync_copy(at_hbm, atv)
    pltpu.sync_copy(bt_hbm, btv)
    pltpu.sync_copy(pv_hbm, pvv)
    iota = lax.broadcasted_iota(jnp.int32, (_L,), 0)
    negv = jnp.full((_L,), _DEAD, jnp.float32)
    bigv = jnp.full((_L,), 1 << 30, jnp.int32)

    def compact_both(rl, t1, t2):
        def comp(j, c):
            oa, ob = c
            for u in range(2):
                basea = 4 * _L * j + 2 * _L * u
                baseb = basea + _PAD
                va1 = hbuf[rl, pl.ds(basea, _L)]
                va2 = hbuf[rl, pl.ds(basea + _L, _L)]
                vb1 = hbuf[rl, pl.ds(baseb, _L)]
                vb2 = hbuf[rl, pl.ds(baseb + _L, _L)]
                ma1 = va1 >= t1
                ma2 = va2 >= t1
                mb1 = vb1 >= t2
                mb2 = vb2 >= t2
                ix = iota + basea
                plsc.store_compressed(sval.at[pl.ds(oa, _L)], va1, mask=ma1)
                plsc.store_compressed(sidx.at[pl.ds(oa, _L)], ix, mask=ma1)
                plsc.store_compressed(sval2.at[pl.ds(ob, _L)], vb1, mask=mb1)
                plsc.store_compressed(sidx2.at[pl.ds(ob, _L)], ix, mask=mb1)
                oa1 = oa + plsc.all_reduce_population_count(ma1)[0]
                ob1 = ob + plsc.all_reduce_population_count(mb1)[0]
                plsc.store_compressed(sval.at[pl.ds(oa1, _L)], va2, mask=ma2)
                plsc.store_compressed(sidx.at[pl.ds(oa1, _L)], ix + _L,
                                      mask=ma2)
                plsc.store_compressed(sval2.at[pl.ds(ob1, _L)], vb2, mask=mb2)
                plsc.store_compressed(sidx2.at[pl.ds(ob1, _L)], ix + _L,
                                      mask=mb2)
                oa = oa1 + plsc.all_reduce_population_count(ma2)[0]
                ob = ob1 + plsc.all_reduce_population_count(mb2)[0]
            return (oa, ob)

        oa, ob = lax.fori_loop(0, _PAD // (4 * _L), comp,
                               (jnp.int32(0), jnp.int32(0)))
        sval[pl.ds(oa, _L)] = negv
        sidx[pl.ds(oa, _L)] = bigv
        sval[pl.ds(oa + _L, _L)] = negv
        sidx[pl.ds(oa + _L, _L)] = bigv
        sval2[pl.ds(ob, _L)] = negv
        sidx2[pl.ds(ob, _L)] = bigv
        sval2[pl.ds(ob + _L, _L)] = negv
        sidx2[pl.ds(ob + _L, _L)] = bigv
        return oa, ob

    def merge_both(oa, ob):
        npa = (oa + 2 * _L - 1) // (2 * _L)
        npb = (ob + 2 * _L - 1) // (2 * _L)
        npmax = jnp.maximum(npa, npb)

        def one(vref, iref, b, r):
            c1k, c1v = _sortkv(vref[pl.ds(b, _L)], iref[pl.ds(b, _L)])
            c2k, c2v = _sortkv(vref[pl.ds(b + _L, _L)],
                               iref[pl.ds(b + _L, _L)])
            t1k, t1v, t2k, t2v = _merge16_full(c1k, c1v, c2k, c2v)
            return _merge32_top(*r, t1k, t1v, t2k, t2v)

        def mstep(j, c):
            ra, rb = c[:4], c[4:]
            b = 2 * _L * j
            na = one(sval, sidx, b, ra)
            nb = one(sval2, sidx2, b, rb)
            ina = j < npa
            inb = j < npb
            ra = tuple(jnp.where(ina, n, o) for n, o in zip(na, ra))
            rb = tuple(jnp.where(inb, n, o) for n, o in zip(nb, rb))
            return ra + rb

        init = (negv, bigv, negv, bigv)
        return lax.fori_loop(0, npmax, mstep, init + init)

    def row_body(r_glob):
        rl = r_glob % _RB
        tv = tbuf[pl.ds(2 * rl, _L)]
        oa, ob = compact_both(rl, tv[0], tv[1])
        (w1a, p1a, w1b, p1b,
         w2a, p2a, w2b, p2b) = merge_both(oa, ob)
        w12[pl.ds(0, _L)] = w1a
        w12[pl.ds(_L, _L)] = w1b
        w12[pl.ds(2 * _L, _L)] = w2a
        w12[pl.ds(3 * _L, _L)] = w2b
        i12[pl.ds(0, _L)] = p1a
        i12[pl.ds(_L, _L)] = p1b
        i12[pl.ds(2 * _L, _L)] = p2a
        i12[pl.ds(3 * _L, _L)] = p2b

        def cchunk(jj):
            ai = atv[pl.ds(_L * jj, _L)]
            bi = btv[pl.ds(_L * jj, _L)]
            pv = pvv[pl.ds(_L * jj, _L)]
            ga = plsc.load_gather(w12, [ai])
            gb = plsc.load_gather(w12, [bi])
            ia = plsc.load_gather(i12, [ai])
            ib = plsc.load_gather(i12, [bi])
            ck = jnp.maximum(ga + gb, 0.0) + pv
            cv = ia * _PKM + ib
            return _sortkv(ck, cv)

        def cpair(j):
            c1k, c1v = cchunk(2 * j)
            c2k, c2v = cchunk(2 * j + 1)
            return _merge16_full(c1k, c1v, c2k, c2v)

        t0, t1_, t2_, t3 = cpair(0), cpair(1), cpair(2), cpair(3)
        r01 = _merge32_top(*t0, *t1_)
        r23 = _merge32_top(*t2_, *t3)
        r1, v1, r2, v2 = _merge32_top(*r01, *r23)

        out_off = r_glob * _K
        wout[pl.ds(out_off, _L)] = r1
        wout[pl.ds(out_off + _L, _L)] = r2
        iout[pl.ds(out_off, _L)] = v1
        iout[pl.ds(out_off + _L, _L)] = v2

    def batch_body(b, _):
        pltpu.sync_copy(h_hbm.at[pl.ds(row0 + b * _RB, _RB), :], hbuf)
        pltpu.sync_copy(
            t_hbm.at[pl.ds((row0 + b * _RB) * 2, _RB * 2)],
            tbuf.at[pl.ds(0, _RB * 2)])

        def rloop(r, _2):
            row_body(b * _RB + r)
            return 0

        lax.fori_loop(0, _RB, rloop, 0)
        return 0

    lax.fori_loop(0, nbatch, batch_body, 0)
    pltpu.sync_copy(wout, ow_hbm.at[pl.ds(row0 * _K, rpw * _K)])
    pltpu.sync_copy(iout, oi_hbm.at[pl.ds(row0 * _K, rpw * _K)])
  return _sc_body


def _sc_topk(h2d, t_flat, atab, btab, padv, ntok):
    rpw = ntok // _NW
    nbatch = rpw // _RB
    mesh = plsc.VectorSubcoreMesh(core_axis_name="c", subcore_axis_name="s",
                                  num_cores=_NC, num_subcores=_NS)
    f = pl.kernel(
        _make_sc_body(rpw, nbatch),
        out_type=(
            jax.ShapeDtypeStruct((ntok * _K,), jnp.float32),
            jax.ShapeDtypeStruct((ntok * _K,), jnp.int32),
        ),
        mesh=mesh,
        compiler_params=pltpu.CompilerParams(needs_layout_passes=False),
        scratch_types=[
            pltpu.VMEM((_RB, 2 * _PAD), jnp.float32),
            pltpu.VMEM((_RB * 2 + _L,), jnp.float32),
            pltpu.VMEM((_PAD + 2 * _L,), jnp.float32),
            pltpu.VMEM((_PAD + 2 * _L,), jnp.int32),
            pltpu.VMEM((_PAD + 2 * _L,), jnp.float32),
            pltpu.VMEM((_PAD + 2 * _L,), jnp.int32),
            pltpu.VMEM((4 * _L,), jnp.float32),
            pltpu.VMEM((4 * _L,), jnp.int32),
            pltpu.VMEM((_NCAND,), jnp.int32),
            pltpu.VMEM((_NCAND,), jnp.int32),
            pltpu.VMEM((_NCAND,), jnp.float32),
            pltpu.VMEM((rpw * _K,), jnp.float32),
            pltpu.VMEM((rpw * _K,), jnp.int32),
        ],
    )
    return f(h2d, t_flat, atab, btab, padv)


def _sc_tables():
    import numpy as np
    at = np.zeros((_NCAND,), np.int32)
    bt = np.zeros((_NCAND,), np.int32)
    pv = np.full((_NCAND,), _DEAD, np.float32)
    for j, (a, b) in enumerate(_AB):
        at[j] = a
        bt[j] = b + _K
        pv[j] = 0.0
    return jnp.asarray(at), jnp.asarray(bt), jnp.asarray(pv)


def kernel(x, W, b_lin, bias, k):
    del bias
    npad = _PAD - _PKM
    zrows = jnp.zeros((npad, _D_IN), W.dtype)
    wpt = jnp.concatenate([W[:_PKM], zrows, W[_PKM:], zrows], axis=0).T
    negs = jnp.full((npad,), _NEG_PAD, jnp.float32)
    bp = jnp.concatenate(
        [b_lin[:_PKM], negs, b_lin[_PKM:], negs]).reshape(1, 2 * _PAD)
    atab, btab, padv = _sc_tables()
    ws, is_ = [], []
    nchunk = _N_TOK // _NSPLIT
    for c in range(_NSPLIT):
        xc = x[c * nchunk:(c + 1) * nchunk]
        h, tt = _matmul(xc, wpt, bp)
        w_flat, i_flat = _sc_topk(h, tt.reshape(-1),
                                  atab, btab, padv, nchunk)
        ws.append(w_flat.reshape(nchunk, _K))
        is_.append(i_flat.reshape(nchunk, _K))
    w = jnp.concatenate(ws, axis=0)
    i = jnp.concatenate(is_, axis=0)
    keep = jnp.asarray(k) == _K
    w = jnp.where(keep, w, jnp.zeros_like(w))
    i = jnp.where(keep, i, jnp.zeros_like(i))
    return w, i


# speedup vs baseline: 1.3285x; 1.0073x over previous
import functools

import jax
import jax.numpy as jnp
from jax import lax
from jax.experimental import pallas as pl
from jax.experimental.pallas import tpu as pltpu
from jax.experimental.pallas import tpu_sc as plsc

_D_IN = 2048
_PKM = 1000
_PAD = 1024
_K = 32
_N_TOK = 8192
_BLK = 256
_NEG_PAD = -1e30
_DEAD = -3e38

_NC = 2
_NS = 16
_L = 16
_NW = _NC * _NS
_RPW = _N_TOK // _NW
_RB = 32
_NBATCH = _RPW // _RB

_AB = [(a, b) for a in range(_K) for b in range(_K) if (a + 1) * (b + 1) <= _K]
_NCAND = 128
_NSPLIT = 1


def _mm_body(x_ref, w_ref, b_ref, h_ref, t_ref):
    h = jnp.dot(x_ref[...], w_ref[...], preferred_element_type=jnp.float32)
    h = h + b_ref[...]
    h_ref[...] = h

    def thresh(v):
        w = _PAD
        while w > _K:
            w //= 2
            v = jnp.maximum(v[:, :w], v[:, w:])
        return jnp.min(v, axis=1, keepdims=True)

    t_ref[...] = jnp.concatenate(
        [thresh(h[:, :_PAD]), thresh(h[:, _PAD:])], axis=1)


def _matmul(x, wpt, bp):
    ntok = x.shape[0]
    fixed = lambda i: (0, 0)
    return pl.pallas_call(
        _mm_body,
        grid=(ntok // _BLK,),
        in_specs=[
            pl.BlockSpec((_BLK, _D_IN), lambda i: (i, 0)),
            pl.BlockSpec((_D_IN, 2 * _PAD), fixed),
            pl.BlockSpec((1, 2 * _PAD), fixed),
        ],
        out_specs=[
            pl.BlockSpec((_BLK, 2 * _PAD), lambda i: (i, 0)),
            pl.BlockSpec((_BLK, 2), lambda i: (i, 0)),
        ],
        out_shape=[
            jax.ShapeDtypeStruct((ntok, 2 * _PAD), jnp.float32),
            jax.ShapeDtypeStruct((ntok, 2), jnp.float32),
        ],
        compiler_params=pltpu.CompilerParams(
            dimension_semantics=("parallel",),
        ),
    )(x, wpt, bp)


def _sortkv(keys, vals):
    return plsc.sort_key_val(keys, vals, descending=True)


def _minmax_kv(ak, av, bk, bv):
    m = ak >= bk
    hk = jnp.where(m, ak, bk)
    hv = jnp.where(m, av, bv)
    lk = jnp.where(m, bk, ak)
    lv = jnp.where(m, bv, av)
    return hk, hv, lk, lv


def _merge16_full(c1k, c1v, c2k, c2v):
    r2k = lax.rev(c2k, (0,))
    r2v = lax.rev(c2v, (0,))
    hk, hv, lk, lv = _minmax_kv(c1k, c1v, r2k, r2v)
    t1k, t1v = _sortkv(hk, hv)
    t2k, t2v = _sortkv(lk, lv)
    return t1k, t1v, t2k, t2v


def _merge32_top(r1k, r1v, r2k, r2v, t1k, t1v, t2k, t2v):
    x1k, x1v, _, _ = _minmax_kv(r1k, r1v, lax.rev(t2k, (0,)),
                                lax.rev(t2v, (0,)))
    x2k, x2v, _, _ = _minmax_kv(r2k, r2v, lax.rev(t1k, (0,)),
                                lax.rev(t1v, (0,)))
    y1k, y1v, y2k, y2v = _minmax_kv(x1k, x1v, x2k, x2v)
    r1k, r1v = _sortkv(y1k, y1v)
    r2k, r2v = _sortkv(y2k, y2v)
    return r1k, r1v, r2k, r2v


def _make_sc_body(rpw, nbatch):
  def _sc_body(h_hbm, t_hbm, at_hbm, bt_hbm, pv_hbm, ow_hbm, oi_hbm,
               hbuf, tbuf, sidx, sidx2, w12, i12, atv, btv, pvv,
               wout, iout):
    wid = lax.axis_index("s") * _NC + lax.axis_index("c")
    row0 = wid * rpw
    pltpu.sync_copy(at_hbm, atv)
    pltpu.sync_copy(bt_hbm, btv)
    pltpu.sync_copy(pv_hbm, pvv)
    iota = lax.broadcasted_iota(jnp.int32, (_L,), 0)
    negv = jnp.full((_L,), _DEAD, jnp.float32)
    bigv = jnp.full((_L,), 1 << 30, jnp.int32)

    def compact_both(rl, t1, t2):
        def comp(j, c):
            oa, ob = c
            for u in range(2):
                basea = 4 * _L * j + 2 * _L * u
                baseb = basea + _PAD
                va1 = hbuf[rl, pl.ds(basea, _L)]
                va2 = hbuf[rl, pl.ds(basea + _L, _L)]
                vb1 = hbuf[rl, pl.ds(baseb, _L)]
                vb2 = hbuf[rl, pl.ds(baseb + _L, _L)]
                ma1 = va1 >= t1
                ma2 = va2 >= t1
                mb1 = vb1 >= t2
                mb2 = vb2 >= t2
                ix = iota + basea
                plsc.store_compressed(sidx.at[pl.ds(oa, _L)], ix, mask=ma1)
                plsc.store_compressed(sidx2.at[pl.ds(ob, _L)], ix, mask=mb1)
                oa1 = oa + plsc.all_reduce_population_count(ma1)[0]
                ob1 = ob + plsc.all_reduce_population_count(mb1)[0]
                plsc.store_compressed(sidx.at[pl.ds(oa1, _L)], ix + _L,
                                      mask=ma2)
                plsc.store_compressed(sidx2.at[pl.ds(ob1, _L)], ix + _L,
                                      mask=mb2)
                oa = oa1 + plsc.all_reduce_population_count(ma2)[0]
                ob = ob1 + plsc.all_reduce_population_count(mb2)[0]
            return (oa, ob)

        oa, ob = lax.fori_loop(0, _PAD // (4 * _L), comp,
                               (jnp.int32(0), jnp.int32(0)))
        padv_i = jnp.full((_L,), _PAD - 1, jnp.int32)
        sidx[pl.ds(oa, _L)] = padv_i
        sidx[pl.ds(oa + _L, _L)] = padv_i
        sidx2[pl.ds(ob, _L)] = padv_i
        sidx2[pl.ds(ob + _L, _L)] = padv_i
        return oa, ob

    def merge_both(rl, oa, ob):
        npa = (oa + 2 * _L - 1) // (2 * _L)
        npb = (ob + 2 * _L - 1) // (2 * _L)
        npmax = jnp.maximum(npa, npb)

        rls = jnp.zeros((_L,), jnp.int32) + rl

        def one(iref, coloff, b, r):
            i1 = iref[pl.ds(b, _L)] & (_PAD - 1)
            i2 = iref[pl.ds(b + _L, _L)] & (_PAD - 1)
            v1 = plsc.load_gather(hbuf, [rls, i1 + coloff])
            v2 = plsc.load_gather(hbuf, [rls, i2 + coloff])
            c1k, c1v = _sortkv(v1, i1)
            c2k, c2v = _sortkv(v2, i2)
            t1k, t1v, t2k, t2v = _merge16_full(c1k, c1v, c2k, c2v)
            return _merge32_top(*r, t1k, t1v, t2k, t2v)

        def mstep(j, c):
            ra, rb = c[:4], c[4:]
            b = 2 * _L * j
            na = one(sidx, 0, b, ra)
            nb = one(sidx2, _PAD, b, rb)
            ina = j < npa
            inb = j < npb
            ra = tuple(jnp.where(ina, n, o) for n, o in zip(na, ra))
            rb = tuple(jnp.where(inb, n, o) for n, o in zip(nb, rb))
            return ra + rb

        init = (negv, bigv, negv, bigv)
        return lax.fori_loop(0, npmax, mstep, init + init)

    def row_body(r_glob):
        rl = r_glob % _RB
        tv = tbuf[pl.ds(2 * rl, _L)]
        oa, ob = compact_both(rl, tv[0], tv[1])
        (w1a, p1a, w1b, p1b,
         w2a, p2a, w2b, p2b) = merge_both(rl, oa, ob)
        w12[pl.ds(0, _L)] = w1a
        w12[pl.ds(_L, _L)] = w1b
        w12[pl.ds(2 * _L, _L)] = w2a
        w12[pl.ds(3 * _L, _L)] = w2b
        i12[pl.ds(0, _L)] = p1a
        i12[pl.ds(_L, _L)] = p1b
        i12[pl.ds(2 * _L, _L)] = p2a
        i12[pl.ds(3 * _L, _L)] = p2b

        def cchunk(jj):
            ai = atv[pl.ds(_L * jj, _L)]
            bi = btv[pl.ds(_L * jj, _L)]
            pv = pvv[pl.ds(_L * jj, _L)]
            ga = plsc.load_gather(w12, [ai])
            gb = plsc.load_gather(w12, [bi])
            ia = plsc.load_gather(i12, [ai])
            ib = plsc.load_gather(i12, [bi])
            ck = jnp.maximum(ga + gb, 0.0) + pv
            cv = ia * _PKM + ib
            return _sortkv(ck, cv)

        def cpair(j):
            c1k, c1v = cchunk(2 * j)
            c2k, c2v = cchunk(2 * j + 1)
            return _merge16_full(c1k, c1v, c2k, c2v)

        t0, t1_, t2_, t3 = cpair(0), cpair(1), cpair(2), cpair(3)
        r01 = _merge32_top(*t0, *t1_)
        r23 = _merge32_top(*t2_, *t3)
        r1, v1, r2, v2 = _merge32_top(*r01, *r23)

        out_off = r_glob * _K
        wout[pl.ds(out_off, _L)] = r1
        wout[pl.ds(out_off + _L, _L)] = r2
        iout[pl.ds(out_off, _L)] = v1
        iout[pl.ds(out_off + _L, _L)] = v2

    def batch_body(b, _):
        pltpu.sync_copy(h_hbm.at[pl.ds(row0 + b * _RB, _RB), :], hbuf)
        pltpu.sync_copy(
            t_hbm.at[pl.ds((row0 + b * _RB) * 2, _RB * 2)],
            tbuf.at[pl.ds(0, _RB * 2)])

        def rloop(r, _2):
            row_body(b * _RB + r)
            return 0

        lax.fori_loop(0, _RB, rloop, 0)
        return 0

    lax.fori_loop(0, nbatch, batch_body, 0)
    pltpu.sync_copy(wout, ow_hbm.at[pl.ds(row0 * _K, rpw * _K)])
    pltpu.sync_copy(iout, oi_hbm.at[pl.ds(row0 * _K, rpw * _K)])
  return _sc_body


def _sc_topk(h2d, t_flat, atab, btab, padv, ntok):
    rpw = ntok // _NW
    nbatch = rpw // _RB
    mesh = plsc.VectorSubcoreMesh(core_axis_name="c", subcore_axis_name="s",
                                  num_cores=_NC, num_subcores=_NS)
    f = pl.kernel(
        _make_sc_body(rpw, nbatch),
        out_type=(
            jax.ShapeDtypeStruct((ntok * _K,), jnp.float32),
            jax.ShapeDtypeStruct((ntok * _K,), jnp.int32),
        ),
        mesh=mesh,
        compiler_params=pltpu.CompilerParams(needs_layout_passes=False),
        scratch_types=[
            pltpu.VMEM((_RB, 2 * _PAD), jnp.float32),
            pltpu.VMEM((_RB * 2 + _L,), jnp.float32),
            pltpu.VMEM((_PAD + 2 * _L,), jnp.int32),
            pltpu.VMEM((_PAD + 2 * _L,), jnp.int32),
            pltpu.VMEM((4 * _L,), jnp.float32),
            pltpu.VMEM((4 * _L,), jnp.int32),
            pltpu.VMEM((_NCAND,), jnp.int32),
            pltpu.VMEM((_NCAND,), jnp.int32),
            pltpu.VMEM((_NCAND,), jnp.float32),
            pltpu.VMEM((rpw * _K,), jnp.float32),
            pltpu.VMEM((rpw * _K,), jnp.int32),
        ],
    )
    return f(h2d, t_flat, atab, btab, padv)


def _sc_tables():
    import numpy as np
    at = np.zeros((_NCAND,), np.int32)
    bt = np.zeros((_NCAND,), np.int32)
    pv = np.full((_NCAND,), _DEAD, np.float32)
    for j, (a, b) in enumerate(_AB):
        at[j] = a
        bt[j] = b + _K
        pv[j] = 0.0
    return jnp.asarray(at), jnp.asarray(bt), jnp.asarray(pv)


def kernel(x, W, b_lin, bias, k):
    del bias
    npad = _PAD - _PKM
    zrows = jnp.zeros((npad, _D_IN), W.dtype)
    wpt = jnp.concatenate([W[:_PKM], zrows, W[_PKM:], zrows], axis=0).T
    negs = jnp.full((npad,), _NEG_PAD, jnp.float32)
    bp = jnp.concatenate(
        [b_lin[:_PKM], negs, b_lin[_PKM:], negs]).reshape(1, 2 * _PAD)
    atab, btab, padv = _sc_tables()
    ws, is_ = [], []
    nchunk = _N_TOK // _NSPLIT
    for c in range(_NSPLIT):
        xc = x[c * nchunk:(c + 1) * nchunk]
        h, tt = _matmul(xc, wpt, bp)
        w_flat, i_flat = _sc_topk(h, tt.reshape(-1),
                                  atab, btab, padv, nchunk)
        ws.append(w_flat.reshape(nchunk, _K))
        is_.append(i_flat.reshape(nchunk, _K))
    w = jnp.concatenate(ws, axis=0)
    i = jnp.concatenate(is_, axis=0)
    keep = jnp.asarray(k) == _K
    w = jnp.where(keep, w, jnp.zeros_like(w))
    i = jnp.where(keep, i, jnp.zeros_like(i))
    return w, i


# speedup vs baseline: 1.3326x; 1.0031x over previous
import functools

import jax
import jax.numpy as jnp
from jax import lax
from jax.experimental import pallas as pl
from jax.experimental.pallas import tpu as pltpu
from jax.experimental.pallas import tpu_sc as plsc

_D_IN = 2048
_PKM = 1000
_PAD = 1024
_K = 32
_N_TOK = 8192
_BLK = 512
_NEG_PAD = -1e30
_DEAD = -3e38

_NC = 2
_NS = 16
_L = 16
_NW = _NC * _NS
_RPW = _N_TOK // _NW
_RB = 32
_NBATCH = _RPW // _RB

_AB = [(a, b) for a in range(_K) for b in range(_K) if (a + 1) * (b + 1) <= _K]
_NCAND = 128
_NSPLIT = 1


def _mm_body(x_ref, w_ref, b_ref, h_ref, t_ref):
    h = jnp.dot(x_ref[...], w_ref[...], preferred_element_type=jnp.float32)
    h = h + b_ref[...]
    h_ref[...] = h

    def thresh(v):
        w = _PAD
        while w > _K:
            w //= 2
            v = jnp.maximum(v[:, :w], v[:, w:])
        return jnp.min(v, axis=1, keepdims=True)

    t_ref[...] = jnp.concatenate(
        [thresh(h[:, :_PAD]), thresh(h[:, _PAD:])], axis=1)


def _matmul(x, wpt, bp):
    ntok = x.shape[0]
    fixed = lambda i: (0, 0)
    return pl.pallas_call(
        _mm_body,
        grid=(ntok // _BLK,),
        in_specs=[
            pl.BlockSpec((_BLK, _D_IN), lambda i: (i, 0)),
            pl.BlockSpec((_D_IN, 2 * _PAD), fixed),
            pl.BlockSpec((1, 2 * _PAD), fixed),
        ],
        out_specs=[
            pl.BlockSpec((_BLK, 2 * _PAD), lambda i: (i, 0)),
            pl.BlockSpec((_BLK, 2), lambda i: (i, 0)),
        ],
        out_shape=[
            jax.ShapeDtypeStruct((ntok, 2 * _PAD), jnp.float32),
            jax.ShapeDtypeStruct((ntok, 2), jnp.float32),
        ],
        compiler_params=pltpu.CompilerParams(
            dimension_semantics=("parallel",),
        ),
    )(x, wpt, bp)


def _sortkv(keys, vals):
    return plsc.sort_key_val(keys, vals, descending=True)


def _minmax_kv(ak, av, bk, bv):
    m = ak >= bk
    hk = jnp.where(m, ak, bk)
    hv = jnp.where(m, av, bv)
    lk = jnp.where(m, bk, ak)
    lv = jnp.where(m, bv, av)
    return hk, hv, lk, lv


def _merge16_full(c1k, c1v, c2k, c2v):
    r2k = lax.rev(c2k, (0,))
    r2v = lax.rev(c2v, (0,))
    hk, hv, lk, lv = _minmax_kv(c1k, c1v, r2k, r2v)
    t1k, t1v = _sortkv(hk, hv)
    t2k, t2v = _sortkv(lk, lv)
    return t1k, t1v, t2k, t2v


def _merge32_top(r1k, r1v, r2k, r2v, t1k, t1v, t2k, t2v):
    x1k, x1v, _, _ = _minmax_kv(r1k, r1v, lax.rev(t2k, (0,)),
                                lax.rev(t2v, (0,)))
    x2k, x2v, _, _ = _minmax_kv(r2k, r2v, lax.rev(t1k, (0,)),
                                lax.rev(t1v, (0,)))
    y1k, y1v, y2k, y2v = _minmax_kv(x1k, x1v, x2k, x2v)
    r1k, r1v = _sortkv(y1k, y1v)
    r2k, r2v = _sortkv(y2k, y2v)
    return r1k, r1v, r2k, r2v


def _make_sc_body(rpw, nbatch):
  def _sc_body(h_hbm, t_hbm, at_hbm, bt_hbm, pv_hbm, ow_hbm, oi_hbm,
               hbuf, tbuf, sidx, sidx2, w12, i12, atv, btv, pvv,
               wout, iout):
    wid = lax.axis_index("s") * _NC + lax.axis_index("c")
    row0 = wid * rpw
    pltpu.sync_copy(at_hbm, atv)
    pltpu.sync_copy(bt_hbm, btv)
    pltpu.sync_copy(pv_hbm, pvv)
    iota = lax.broadcasted_iota(jnp.int32, (_L,), 0)
    negv = jnp.full((_L,), _DEAD, jnp.float32)
    bigv = jnp.full((_L,), 1 << 30, jnp.int32)

    def compact_both(rl, t1, t2):
        def comp(j, c):
            oa, ob = c
            for u in range(2):
                basea = 4 * _L * j + 2 * _L * u
                baseb = basea + _PAD
                va1 = hbuf[rl, pl.ds(basea, _L)]
                va2 = hbuf[rl, pl.ds(basea + _L, _L)]
                vb1 = hbuf[rl, pl.ds(baseb, _L)]
                vb2 = hbuf[rl, pl.ds(baseb + _L, _L)]
                ma1 = va1 >= t1
                ma2 = va2 >= t1
                mb1 = vb1 >= t2
                mb2 = vb2 >= t2
                ix = iota + basea
                plsc.store_compressed(sidx.at[pl.ds(oa, _L)], ix, mask=ma1)
                plsc.store_compressed(sidx2.at[pl.ds(ob, _L)], ix, mask=mb1)
                oa1 = oa + plsc.all_reduce_population_count(ma1)[0]
                ob1 = ob + plsc.all_reduce_population_count(mb1)[0]
                plsc.store_compressed(sidx.at[pl.ds(oa1, _L)], ix + _L,
                                      mask=ma2)
                plsc.store_compressed(sidx2.at[pl.ds(ob1, _L)], ix + _L,
                                      mask=mb2)
                oa = oa1 + plsc.all_reduce_population_count(ma2)[0]
                ob = ob1 + plsc.all_reduce_population_count(mb2)[0]
            return (oa, ob)

        oa, ob = lax.fori_loop(0, _PAD // (4 * _L), comp,
                               (jnp.int32(0), jnp.int32(0)))
        padv_i = jnp.full((_L,), _PAD - 1, jnp.int32)
        sidx[pl.ds(oa, _L)] = padv_i
        sidx[pl.ds(oa + _L, _L)] = padv_i
        sidx2[pl.ds(ob, _L)] = padv_i
        sidx2[pl.ds(ob + _L, _L)] = padv_i
        return oa, ob

    def merge_both(rl, oa, ob):
        npa = (oa + 2 * _L - 1) // (2 * _L)
        npb = (ob + 2 * _L - 1) // (2 * _L)
        npmax = jnp.maximum(npa, npb)

        rls = jnp.zeros((_L,), jnp.int32) + rl

        def one(iref, coloff, b, r):
            i1 = iref[pl.ds(b, _L)] & (_PAD - 1)
            i2 = iref[pl.ds(b + _L, _L)] & (_PAD - 1)
            v1 = plsc.load_gather(hbuf, [rls, i1 + coloff])
            v2 = plsc.load_gather(hbuf, [rls, i2 + coloff])
            c1k, c1v = _sortkv(v1, i1)
            c2k, c2v = _sortkv(v2, i2)
            t1k, t1v, t2k, t2v = _merge16_full(c1k, c1v, c2k, c2v)
            return _merge32_top(*r, t1k, t1v, t2k, t2v)

        def mstep(j, c):
            ra, rb = c[:4], c[4:]
            b = 2 * _L * j
            na = one(sidx, 0, b, ra)
            nb = one(sidx2, _PAD, b, rb)
            ina = j < npa
            inb = j < npb
            ra = tuple(jnp.where(ina, n, o) for n, o in zip(na, ra))
            rb = tuple(jnp.where(inb, n, o) for n, o in zip(nb, rb))
            return ra + rb

        init = (negv, bigv, negv, bigv)
        return lax.fori_loop(0, npmax, mstep, init + init)

    def row_body(r_glob):
        rl = r_glob % _RB
        tv = tbuf[pl.ds(2 * rl, _L)]
        oa, ob = compact_both(rl, tv[0], tv[1])
        (w1a, p1a, w1b, p1b,
         w2a, p2a, w2b, p2b) = merge_both(rl, oa, ob)
        w12[pl.ds(0, _L)] = w1a
        w12[pl.ds(_L, _L)] = w1b
        w12[pl.ds(2 * _L, _L)] = w2a
        w12[pl.ds(3 * _L, _L)] = w2b
        i12[pl.ds(0, _L)] = p1a
        i12[pl.ds(_L, _L)] = p1b
        i12[pl.ds(2 * _L, _L)] = p2a
        i12[pl.ds(3 * _L, _L)] = p2b

        def cchunk(jj):
            ai = atv[pl.ds(_L * jj, _L)]
            bi = btv[pl.ds(_L * jj, _L)]
            pv = pvv[pl.ds(_L * jj, _L)]
            ga = plsc.load_gather(w12, [ai])
            gb = plsc.load_gather(w12, [bi])
            ia = plsc.load_gather(i12, [ai])
            ib = plsc.load_gather(i12, [bi])
            ck = jnp.maximum(ga + gb, 0.0) + pv
            cv = ia * _PKM + ib
            return _sortkv(ck, cv)

        def cpair(j):
            c1k, c1v = cchunk(2 * j)
            c2k, c2v = cchunk(2 * j + 1)
            return _merge16_full(c1k, c1v, c2k, c2v)

        t0, t1_, t2_, t3 = cpair(0), cpair(1), cpair(2), cpair(3)
        r01 = _merge32_top(*t0, *t1_)
        r23 = _merge32_top(*t2_, *t3)
        r1, v1, r2, v2 = _merge32_top(*r01, *r23)

        out_off = r_glob * _K
        wout[pl.ds(out_off, _L)] = r1
        wout[pl.ds(out_off + _L, _L)] = r2
        iout[pl.ds(out_off, _L)] = v1
        iout[pl.ds(out_off + _L, _L)] = v2

    def batch_body(b, _):
        pltpu.sync_copy(h_hbm.at[pl.ds(row0 + b * _RB, _RB), :], hbuf)
        pltpu.sync_copy(
            t_hbm.at[pl.ds((row0 + b * _RB) * 2, _RB * 2)],
            tbuf.at[pl.ds(0, _RB * 2)])

        def rloop(r, _2):
            row_body(b * _RB + r)
            return 0

        lax.fori_loop(0, _RB, rloop, 0)
        return 0

    lax.fori_loop(0, nbatch, batch_body, 0)
    pltpu.sync_copy(wout, ow_hbm.at[pl.ds(row0 * _K, rpw * _K)])
    pltpu.sync_copy(iout, oi_hbm.at[pl.ds(row0 * _K, rpw * _K)])
  return _sc_body


def _sc_topk(h2d, t_flat, atab, btab, padv, ntok):
    rpw = ntok // _NW
    nbatch = rpw // _RB
    mesh = plsc.VectorSubcoreMesh(core_axis_name="c", subcore_axis_name="s",
                                  num_cores=_NC, num_subcores=_NS)
    f = pl.kernel(
        _make_sc_body(rpw, nbatch),
        out_type=(
            jax.ShapeDtypeStruct((ntok * _K,), jnp.float32),
            jax.ShapeDtypeStruct((ntok * _K,), jnp.int32),
        ),
        mesh=mesh,
        compiler_params=pltpu.CompilerParams(needs_layout_passes=False),
        scratch_types=[
            pltpu.VMEM((_RB, 2 * _PAD), jnp.float32),
            pltpu.VMEM((_RB * 2 + _L,), jnp.float32),
            pltpu.VMEM((_PAD + 2 * _L,), jnp.int32),
            pltpu.VMEM((_PAD + 2 * _L,), jnp.int32),
            pltpu.VMEM((4 * _L,), jnp.float32),
            pltpu.VMEM((4 * _L,), jnp.int32),
            pltpu.VMEM((_NCAND,), jnp.int32),
            pltpu.VMEM((_NCAND,), jnp.int32),
            pltpu.VMEM((_NCAND,), jnp.float32),
            pltpu.VMEM((rpw * _K,), jnp.float32),
            pltpu.VMEM((rpw * _K,), jnp.int32),
        ],
    )
    return f(h2d, t_flat, atab, btab, padv)


def _sc_tables():
    import numpy as np
    at = np.zeros((_NCAND,), np.int32)
    bt = np.zeros((_NCAND,), np.int32)
    pv = np.full((_NCAND,), _DEAD, np.float32)
    for j, (a, b) in enumerate(_AB):
        at[j] = a
        bt[j] = b + _K
        pv[j] = 0.0
    return jnp.asarray(at), jnp.asarray(bt), jnp.asarray(pv)


def kernel(x, W, b_lin, bias, k):
    del bias
    npad = _PAD - _PKM
    zrows = jnp.zeros((npad, _D_IN), W.dtype)
    wpt = jnp.concatenate([W[:_PKM], zrows, W[_PKM:], zrows], axis=0).T
    negs = jnp.full((npad,), _NEG_PAD, jnp.float32)
    bp = jnp.concatenate(
        [b_lin[:_PKM], negs, b_lin[_PKM:], negs]).reshape(1, 2 * _PAD)
    atab, btab, padv = _sc_tables()
    ws, is_ = [], []
    nchunk = _N_TOK // _NSPLIT
    for c in range(_NSPLIT):
        xc = x[c * nchunk:(c + 1) * nchunk]
        h, tt = _matmul(xc, wpt, bp)
        w_flat, i_flat = _sc_topk(h, tt.reshape(-1),
                                  atab, btab, padv, nchunk)
        ws.append(w_flat.reshape(nchunk, _K))
        is_.append(i_flat.reshape(nchunk, _K))
    w = jnp.concatenate(ws, axis=0)
    i = jnp.concatenate(is_, axis=0)
    keep = jnp.asarray(k) == _K
    w = jnp.where(keep, w, jnp.zeros_like(w))
    i = jnp.where(keep, i, jnp.zeros_like(i))
    return w, i
